# baseline (speedup 1.0000x reference)
import functools
import math

import jax
import jax.numpy as jnp
from jax import lax
from jax.experimental import pallas as pl
from jax.experimental.pallas import tpu as pltpu
from jax.experimental.pallas import tpu_sc as plsc

N = 10000
E = 320000
D = 128
H = 8
L = 3
LANES = 16
NC = 2
NS = 16
NW = NC * NS
EPW = E // NW
C = 80
NCHUNK = EPW // C
GRP = C // LANES
RSTEP = 632

INV_SQRT_D = 1.0 / math.sqrt(D)
BN = 2000
BE = 4000


def _prep_body(x_ref, wq, bq, wk, wv, bv, wl, bl, q_ref, xk_ref, v_ref, lx_ref):
    xb = x_ref[...]
    q_ref[...] = jnp.dot(xb, wq[...], preferred_element_type=jnp.float32) + bq[...]
    xk_ref[...] = jnp.dot(xb, wk[...], preferred_element_type=jnp.float32) * INV_SQRT_D
    v_ref[...] = jnp.dot(xb, wv[...], preferred_element_type=jnp.float32) + bv[...]
    lx_ref[...] = jnp.dot(xb, wl[...], preferred_element_type=jnp.float32) + bl[...]


_prep = pl.pallas_call(
    _prep_body,
    grid=(N // BN,),
    in_specs=[
        pl.BlockSpec((BN, D), lambda i: (i, 0)),
        pl.BlockSpec((D, D), lambda i: (0, 0)),
        pl.BlockSpec((1, D), lambda i: (0, 0)),
        pl.BlockSpec((D, D), lambda i: (0, 0)),
        pl.BlockSpec((D, D), lambda i: (0, 0)),
        pl.BlockSpec((1, D), lambda i: (0, 0)),
        pl.BlockSpec((D, D), lambda i: (0, 0)),
        pl.BlockSpec((1, D), lambda i: (0, 0)),
    ],
    out_specs=[pl.BlockSpec((BN, D), lambda i: (i, 0))] * 4,
    out_shape=[jax.ShapeDtypeStruct((N, D), jnp.float32)] * 4,
)


def _ea_body(ea_ref, w0, b0, w1, b1, w2, b2, o0, o1, o2):
    eb = ea_ref[...]
    o0[...] = (jnp.dot(eb, w0[...], preferred_element_type=jnp.float32) + b0[...]) * INV_SQRT_D
    o1[...] = (jnp.dot(eb, w1[...], preferred_element_type=jnp.float32) + b1[...]) * INV_SQRT_D
    o2[...] = (jnp.dot(eb, w2[...], preferred_element_type=jnp.float32) + b2[...]) * INV_SQRT_D


_ea_prep = pl.pallas_call(
    _ea_body,
    grid=(E // BE,),
    in_specs=[pl.BlockSpec((BE, D), lambda i: (i, 0))]
    + [pl.BlockSpec((D, D), lambda i: (0, 0)), pl.BlockSpec((1, D), lambda i: (0, 0))] * 3,
    out_specs=[pl.BlockSpec((BE, D), lambda i: (i, 0))] * 3,
    out_shape=[jax.ShapeDtypeStruct((E, D), jnp.float32)] * 3,
)


def _comb_body(scp_ref, rsc_ref):
    s = scp_ref[0] + scp_ref[1]
    rsc_ref[...] = 1.0 / jnp.where(s == 0.0, 1.0, s)


_combine = pl.pallas_call(
    _comb_body,
    in_specs=[pl.BlockSpec((NC, N, 16), lambda: (0, 0, 0))],
    out_specs=pl.BlockSpec((N, 16), lambda: (0, 0)),
    out_shape=jax.ShapeDtypeStruct((N, 16), jnp.float32),
)


def _node_body(x_ref, lx_ref, hp_ref, scp_ref, cww, cwb, a1a, a1b, a1bias,
               w20, w21, b20, b21, out_ref):
    xb = x_ref[...]
    hpre = hp_ref[0] + hp_ref[1]
    deg = scp_ref[0, :, 8:9] + scp_ref[1, :, 8:9]
    h = jnp.dot(hpre, cww[...], preferred_element_type=jnp.float32) + deg * cwb[...]
    z = (jnp.dot(lx_ref[...], a1a[...], preferred_element_type=jnp.float32)
         + jnp.dot(h, a1b[...], preferred_element_type=jnp.float32) + a1bias[...])
    z = jnp.where(z > 0, z, 0.2 * z)
    p0 = jnp.sum(z * w20[...], axis=1, keepdims=True) + b20[...]
    p1 = jnp.sum(z * w21[...], axis=1, keepdims=True) + b21[...]
    m = jnp.maximum(p0, p1)
    e0 = jnp.exp(p0 - m)
    e1 = jnp.exp(p1 - m)
    inv = 1.0 / (e0 + e1)
    out_ref[...] = xb * (e0 * inv) + h * (e1 * inv)


_node = pl.pallas_call(
    _node_body,
    grid=(N // BN,),
    in_specs=[
        pl.BlockSpec((BN, D), lambda i: (i, 0)),
        pl.BlockSpec((BN, D), lambda i: (i, 0)),
        pl.BlockSpec((NC, BN, D), lambda i: (0, i, 0)),
        pl.BlockSpec((NC, BN, 16), lambda i: (0, i, 0)),
        pl.BlockSpec((D, D), lambda i: (0, 0)),
        pl.BlockSpec((1, D), lambda i: (0, 0)),
        pl.BlockSpec((D, D), lambda i: (0, 0)),
        pl.BlockSpec((D, D), lambda i: (0, 0)),
        pl.BlockSpec((1, D), lambda i: (0, 0)),
        pl.BlockSpec((1, D), lambda i: (0, 0)),
        pl.BlockSpec((1, D), lambda i: (0, 0)),
        pl.BlockSpec((1, 1), lambda i: (0, 0)),
        pl.BlockSpec((1, 1), lambda i: (0, 0)),
    ],
    out_specs=pl.BlockSpec((BN, D), lambda i: (i, 0)),
    out_shape=jax.ShapeDtypeStruct((N, D), jnp.float32),
)


_sc_mesh = plsc.VectorSubcoreMesh(core_axis_name="c", subcore_axis_name="s")


def _p1_body(q_hbm, xk_hbm, ea_hbm, src_hbm, dst_hbm, awsp_hbm, absp_hbm, z16_hbm,
             a_hbm, scp_hbm,
             sidx, didx, qrows, krows, erows, abuf, awsp_v, absp_v, sc_sh):
    c = lax.axis_index("c")
    s = lax.axis_index("s")
    wid = c * NS + s
    e0 = wid * EPW
    r0 = jnp.minimum(s * RSTEP, N - RSTEP)
    pltpu.sync_copy(z16_hbm.at[pl.ds(r0, RSTEP)], sc_sh.at[pl.ds(r0, RSTEP)])
    pltpu.sync_copy(awsp_hbm, awsp_v)
    pltpu.sync_copy(absp_hbm, absp_v)
    plsc.subcore_barrier()

    iota = lax.iota(jnp.int32, LANES)

    def chunk(ch, carry):
        base = e0 + ch * C
        pltpu.sync_copy(src_hbm.at[pl.ds(base, C)], sidx)
        pltpu.sync_copy(dst_hbm.at[pl.ds(base, C)], didx)
        pltpu.sync_copy(q_hbm.at[didx], qrows)
        pltpu.sync_copy(xk_hbm.at[sidx], krows)
        pltpu.sync_copy(ea_hbm.at[pl.ds(base, C)], erows)

        def group(g, carry2):
            rows = g * LANES + iota
            ls = []
            for h in range(H):
                acc = jnp.zeros((LANES,), jnp.float32)
                for dd in range(LANES):
                    col = jnp.full((LANES,), h * LANES + dd, jnp.int32)
                    qv = plsc.load_gather(qrows, [rows, col])
                    kv = plsc.load_gather(krows, [rows, col])
                    ev = plsc.load_gather(erows, [rows, col])
                    acc = acc + qv * (kv + ev)
                ls.append(jnp.where(acc > 0, acc, 0.2 * acc))
            for j in range(H):
                t = absp_v[j]
                for h in range(H):
                    t = t + ls[h] * awsp_v[h, j]
                aj = jnp.exp(t)
                plsc.store_scatter(abuf, [rows, jnp.full((LANES,), j, jnp.int32)], aj)
            plsc.store_scatter(abuf, [rows, jnp.full((LANES,), H, jnp.int32)],
                               jnp.ones((LANES,), jnp.float32))
            return carry2

        lax.fori_loop(0, GRP, group, 0)
        pltpu.sync_copy(abuf, a_hbm.at[pl.ds(base, C)])
        pltpu.sync_copy(abuf, sc_sh.at[didx], add=True)
        return carry

    lax.fori_loop(0, NCHUNK, chunk, 0)
    plsc.subcore_barrier()
    pltpu.sync_copy(sc_sh.at[pl.ds(r0, RSTEP)],
                    scp_hbm.at[c, pl.ds(r0, RSTEP)])


_p1 = pl.kernel(
    _p1_body,
    out_type=[jax.ShapeDtypeStruct((E, 16), jnp.float32),
              jax.ShapeDtypeStruct((NC, N, 16), jnp.float32)],
    mesh=_sc_mesh,
    compiler_params=pltpu.CompilerParams(needs_layout_passes=False, use_tc_tiling_on_sc=False),
    scratch_types=[
        pltpu.VMEM((C,), jnp.int32),
        pltpu.VMEM((C,), jnp.int32),
        pltpu.VMEM((C, D), jnp.float32),
        pltpu.VMEM((C, D), jnp.float32),
        pltpu.VMEM((C, D), jnp.float32),
        pltpu.VMEM((C, 16), jnp.float32),
        pltpu.VMEM((H, H, LANES), jnp.float32),
        pltpu.VMEM((H, LANES), jnp.float32),
        pltpu.VMEM_SHARED((N, 16), jnp.float32),
    ],
)


def _p2_body(v_hbm, rsc_hbm, a_hbm, src_hbm, dst_hbm, z128_hbm,
             hp_hbm,
             sidx, didx, vrows, arows, rrows, mbuf, h_sh):
    c = lax.axis_index("c")
    s = lax.axis_index("s")
    wid = c * NS + s
    e0 = wid * EPW
    r0 = jnp.minimum(s * RSTEP, N - RSTEP)
    pltpu.sync_copy(z128_hbm.at[pl.ds(r0, RSTEP)], h_sh.at[pl.ds(r0, RSTEP)])
    plsc.subcore_barrier()

    iota = lax.iota(jnp.int32, LANES)

    def chunk(ch, carry):
        base = e0 + ch * C
        pltpu.sync_copy(src_hbm.at[pl.ds(base, C)], sidx)
        pltpu.sync_copy(dst_hbm.at[pl.ds(base, C)], didx)
        pltpu.sync_copy(v_hbm.at[sidx], vrows)
        pltpu.sync_copy(rsc_hbm.at[didx], rrows)
        pltpu.sync_copy(a_hbm.at[pl.ds(base, C)], arows)

        def group(g, carry2):
            rows = g * LANES + iota
            for h in range(H):
                hcol = jnp.full((LANES,), h, jnp.int32)
                attn = (plsc.load_gather(arows, [rows, hcol])
                        * plsc.load_gather(rrows, [rows, hcol]))
                for dd in range(LANES):
                    col = jnp.full((LANES,), h * LANES + dd, jnp.int32)
                    mv = plsc.load_gather(vrows, [rows, col])
                    plsc.store_scatter(mbuf, [rows, col], attn * mv)
            return carry2

        lax.fori_loop(0, GRP, group, 0)
        pltpu.sync_copy(mbuf, h_sh.at[didx], add=True)
        return carry

    lax.fori_loop(0, NCHUNK, chunk, 0)
    plsc.subcore_barrier()
    pltpu.sync_copy(h_sh.at[pl.ds(r0, RSTEP)],
                    hp_hbm.at[c, pl.ds(r0, RSTEP)])


_p2 = pl.kernel(
    _p2_body,
    out_type=jax.ShapeDtypeStruct((NC, N, D), jnp.float32),
    mesh=_sc_mesh,
    compiler_params=pltpu.CompilerParams(needs_layout_passes=False, use_tc_tiling_on_sc=False),
    scratch_types=[
        pltpu.VMEM((C,), jnp.int32),
        pltpu.VMEM((C,), jnp.int32),
        pltpu.VMEM((C, D), jnp.float32),
        pltpu.VMEM((C, 16), jnp.float32),
        pltpu.VMEM((C, 16), jnp.float32),
        pltpu.VMEM((C, D), jnp.float32),
        pltpu.VMEM_SHARED((N, D), jnp.float32),
    ],
)


def kernel(x, edge_index, edge_attr, ck_w, ck_b, qw_w, qw_b, vw_w, vw_b,
           aw_w, aw_b, cw_w, cw_b, lx_w, lx_b, la1_w, la1_b, la2_w, la2_b):
    src = edge_index[0]
    dst = edge_index[1]
    z16 = jnp.zeros((N, 16), jnp.float32)
    z128 = jnp.zeros((N, D), jnp.float32)

    eas = _ea_prep(edge_attr,
                   ck_w[0, :D], ck_b[0].reshape(1, D),
                   ck_w[1, :D], ck_b[1].reshape(1, D),
                   ck_w[2, :D], ck_b[2].reshape(1, D))

    for l in range(L):
        q, xk, v, lxv = _prep(x,
                              qw_w[l], qw_b[l].reshape(1, D),
                              ck_w[l, D:],
                              vw_w[l], vw_b[l].reshape(1, D),
                              lx_w[l], lx_b[l].reshape(1, D))
        awsp = jnp.broadcast_to(aw_w[l][:, :, None], (H, H, LANES))
        absp = jnp.broadcast_to(aw_b[l][:, None], (H, LANES))
        a_e, scp = _p1(q, xk, eas[l], src, dst, awsp, absp, z16)
        rsc = _combine(scp)
        hp = _p2(v, rsc, a_e, src, dst, z128)
        x = _node(x, lxv, hp, scp,
                  cw_w[l], cw_b[l].reshape(1, D),
                  la1_w[l, :D], la1_w[l, D:], la1_b[l].reshape(1, D),
                  la2_w[l, :, 0].reshape(1, D), la2_w[l, :, 1].reshape(1, D),
                  la2_b[l, 0].reshape(1, 1), la2_b[l, 1].reshape(1, 1))
    return x


# speedup vs baseline: 1.1793x; 1.1793x over previous
import functools
import math

import jax
import jax.numpy as jnp
from jax import lax
from jax.experimental import pallas as pl
from jax.experimental.pallas import tpu as pltpu
from jax.experimental.pallas import tpu_sc as plsc

N = 10000
E = 320000
D = 128
H = 8
L = 3
LANES = 16
NC = 2
NS = 16
NW = NC * NS
EPW = E // NW
C = 80
NCHUNK = EPW // C
GRP = C // LANES
RSTEP = 632

INV_SQRT_D = 1.0 / math.sqrt(D)
BN = 2000
BE = 4000


def _prep_body(x_ref, wq, bq, wk, wv, bv, wl, bl, q_ref, xk_ref, v_ref, lx_ref):
    xb = x_ref[...]
    q_ref[...] = jnp.dot(xb, wq[...], preferred_element_type=jnp.float32) + bq[...]
    xk_ref[...] = jnp.dot(xb, wk[...], preferred_element_type=jnp.float32) * INV_SQRT_D
    v_ref[...] = jnp.dot(xb, wv[...], preferred_element_type=jnp.float32) + bv[...]
    lx_ref[...] = jnp.dot(xb, wl[...], preferred_element_type=jnp.float32) + bl[...]


_prep = pl.pallas_call(
    _prep_body,
    grid=(N // BN,),
    in_specs=[
        pl.BlockSpec((BN, D), lambda i: (i, 0)),
        pl.BlockSpec((D, D), lambda i: (0, 0)),
        pl.BlockSpec((1, D), lambda i: (0, 0)),
        pl.BlockSpec((D, D), lambda i: (0, 0)),
        pl.BlockSpec((D, D), lambda i: (0, 0)),
        pl.BlockSpec((1, D), lambda i: (0, 0)),
        pl.BlockSpec((D, D), lambda i: (0, 0)),
        pl.BlockSpec((1, D), lambda i: (0, 0)),
    ],
    out_specs=[pl.BlockSpec((BN, D), lambda i: (i, 0))] * 4,
    out_shape=[jax.ShapeDtypeStruct((N, D), jnp.float32)] * 4,
)


def _ea_body(ea_ref, w0, b0, w1, b1, w2, b2, o0, o1, o2):
    eb = ea_ref[...]
    o0[...] = (jnp.dot(eb, w0[...], preferred_element_type=jnp.float32) + b0[...]) * INV_SQRT_D
    o1[...] = (jnp.dot(eb, w1[...], preferred_element_type=jnp.float32) + b1[...]) * INV_SQRT_D
    o2[...] = (jnp.dot(eb, w2[...], preferred_element_type=jnp.float32) + b2[...]) * INV_SQRT_D


_ea_prep = pl.pallas_call(
    _ea_body,
    grid=(E // BE,),
    in_specs=[pl.BlockSpec((BE, D), lambda i: (i, 0))]
    + [pl.BlockSpec((D, D), lambda i: (0, 0)), pl.BlockSpec((1, D), lambda i: (0, 0))] * 3,
    out_specs=[pl.BlockSpec((BE, D), lambda i: (i, 0))] * 3,
    out_shape=[jax.ShapeDtypeStruct((E, D), jnp.float32)] * 3,
)


def _comb_body(scp_ref, rsc_ref):
    s = scp_ref[0] + scp_ref[1]
    rsc_ref[...] = 1.0 / jnp.where(s == 0.0, 1.0, s)


_combine = pl.pallas_call(
    _comb_body,
    in_specs=[pl.BlockSpec((NC, N, 16), lambda: (0, 0, 0))],
    out_specs=pl.BlockSpec((N, 16), lambda: (0, 0)),
    out_shape=jax.ShapeDtypeStruct((N, 16), jnp.float32),
)


def _node_body(x_ref, lx_ref, hp_ref, scp_ref, cww, cwb, a1a, a1b, a1bias,
               w20, w21, b20, b21, out_ref):
    xb = x_ref[...]
    hpre = hp_ref[0] + hp_ref[1]
    deg = scp_ref[0, :, 8:9] + scp_ref[1, :, 8:9]
    h = jnp.dot(hpre, cww[...], preferred_element_type=jnp.float32) + deg * cwb[...]
    z = (jnp.dot(lx_ref[...], a1a[...], preferred_element_type=jnp.float32)
         + jnp.dot(h, a1b[...], preferred_element_type=jnp.float32) + a1bias[...])
    z = jnp.where(z > 0, z, 0.2 * z)
    p0 = jnp.sum(z * w20[...], axis=1, keepdims=True) + b20[...]
    p1 = jnp.sum(z * w21[...], axis=1, keepdims=True) + b21[...]
    m = jnp.maximum(p0, p1)
    e0 = jnp.exp(p0 - m)
    e1 = jnp.exp(p1 - m)
    inv = 1.0 / (e0 + e1)
    out_ref[...] = xb * (e0 * inv) + h * (e1 * inv)


_node = pl.pallas_call(
    _node_body,
    grid=(N // BN,),
    in_specs=[
        pl.BlockSpec((BN, D), lambda i: (i, 0)),
        pl.BlockSpec((BN, D), lambda i: (i, 0)),
        pl.BlockSpec((NC, BN, D), lambda i: (0, i, 0)),
        pl.BlockSpec((NC, BN, 16), lambda i: (0, i, 0)),
        pl.BlockSpec((D, D), lambda i: (0, 0)),
        pl.BlockSpec((1, D), lambda i: (0, 0)),
        pl.BlockSpec((D, D), lambda i: (0, 0)),
        pl.BlockSpec((D, D), lambda i: (0, 0)),
        pl.BlockSpec((1, D), lambda i: (0, 0)),
        pl.BlockSpec((1, D), lambda i: (0, 0)),
        pl.BlockSpec((1, D), lambda i: (0, 0)),
        pl.BlockSpec((1, 1), lambda i: (0, 0)),
        pl.BlockSpec((1, 1), lambda i: (0, 0)),
    ],
    out_specs=pl.BlockSpec((BN, D), lambda i: (i, 0)),
    out_shape=jax.ShapeDtypeStruct((N, D), jnp.float32),
)


_sc_mesh = plsc.VectorSubcoreMesh(core_axis_name="c", subcore_axis_name="s")


def _p1_body(q_hbm, xk_hbm, ea_hbm, ei_hbm, awsp_hbm, absp_hbm, z16_hbm,
             a_hbm, scp_hbm,
             sdidx, qrows, krows, erows, abuf, awsp_v, absp_v,
             sem_i, sem_r, sem_a, sc_sh):
    c = lax.axis_index("c")
    s = lax.axis_index("s")
    wid = c * NS + s
    e0 = wid * EPW
    r0 = jnp.minimum(s * RSTEP, N - RSTEP)
    pltpu.sync_copy(z16_hbm.at[pl.ds(r0, RSTEP)], sc_sh.at[pl.ds(r0, RSTEP)])
    pltpu.sync_copy(awsp_hbm, awsp_v)
    pltpu.sync_copy(absp_hbm, absp_v)
    plsc.subcore_barrier()

    iota = lax.iota(jnp.int32, LANES)

    def idx_copy(ch, b):
        base = e0 + ch * C
        return pltpu.make_async_copy(ei_hbm.at[:, pl.ds(base, C)], sdidx.at[b],
                                     sem_i.at[b])

    def row_copies(ch, b):
        base = e0 + ch * C
        return (
            pltpu.make_async_copy(q_hbm.at[sdidx.at[b, 1]], qrows.at[b], sem_r.at[b]),
            pltpu.make_async_copy(xk_hbm.at[sdidx.at[b, 0]], krows.at[b], sem_r.at[b]),
            pltpu.make_async_copy(ea_hbm.at[pl.ds(base, C)], erows.at[b], sem_r.at[b]),
        )

    def a_copy(ch, b):
        base = e0 + ch * C
        return pltpu.make_async_copy(abuf.at[b], a_hbm.at[pl.ds(base, C)],
                                     sem_a.at[b])

    def compute(ch, b):
        qr = qrows.at[b]
        kr = krows.at[b]
        er = erows.at[b]

        def group(g, carry2):
            rows = g * LANES + iota
            ls = []
            for h in range(H):
                acc = jnp.zeros((LANES,), jnp.float32)
                for dd in range(LANES):
                    col = jnp.full((LANES,), h * LANES + dd, jnp.int32)
                    qv = plsc.load_gather(qr, [rows, col])
                    kv = plsc.load_gather(kr, [rows, col])
                    ev = plsc.load_gather(er, [rows, col])
                    acc = acc + qv * (kv + ev)
                ls.append(jnp.where(acc > 0, acc, 0.2 * acc))
            for j in range(H):
                t = absp_v[j]
                for h in range(H):
                    t = t + ls[h] * awsp_v[h, j]
                aj = jnp.exp(t)
                plsc.store_scatter(abuf.at[b], [rows, jnp.full((LANES,), j, jnp.int32)], aj)
            plsc.store_scatter(abuf.at[b], [rows, jnp.full((LANES,), H, jnp.int32)],
                               jnp.ones((LANES,), jnp.float32))
            return carry2

        lax.fori_loop(0, GRP, group, 0)
        a_copy(ch, b).start()
        pltpu.sync_copy(abuf.at[b], sc_sh.at[sdidx.at[b, 1]], add=True)

    idx_copy(0, 0).start()

    def step(ch, carry):
        b = lax.rem(ch, 2)
        bb = 1 - b

        @pl.when(ch < NCHUNK)
        def _fetch():
            idx_copy(ch, b).wait()
            for cp in row_copies(ch, b):
                cp.start()

        @pl.when(ch >= 1)
        def _work():
            for cp in row_copies(ch - 1, bb):
                cp.wait()

            @pl.when(ch >= 3)
            def _drain_a():
                a_copy(ch - 3, bb).wait()

            compute(ch - 1, bb)

        @pl.when(ch + 1 < NCHUNK)
        def _prefetch_idx():
            idx_copy(ch + 1, bb).start()

        return carry

    lax.fori_loop(0, NCHUNK + 1, step, 0)
    a_copy(NCHUNK - 2, lax.rem(NCHUNK - 2, 2)).wait()
    a_copy(NCHUNK - 1, lax.rem(NCHUNK - 1, 2)).wait()
    plsc.subcore_barrier()
    pltpu.sync_copy(sc_sh.at[pl.ds(r0, RSTEP)],
                    scp_hbm.at[c, pl.ds(r0, RSTEP)])


_p1 = pl.kernel(
    _p1_body,
    out_type=[jax.ShapeDtypeStruct((E, 16), jnp.float32),
              jax.ShapeDtypeStruct((NC, N, 16), jnp.float32)],
    mesh=_sc_mesh,
    compiler_params=pltpu.CompilerParams(needs_layout_passes=False, use_tc_tiling_on_sc=False),
    scratch_types=[
        pltpu.VMEM((2, 2, C), jnp.int32),
        pltpu.VMEM((2, C, D), jnp.float32),
        pltpu.VMEM((2, C, D), jnp.float32),
        pltpu.VMEM((2, C, D), jnp.float32),
        pltpu.VMEM((2, C, 16), jnp.float32),
        pltpu.VMEM((H, H, LANES), jnp.float32),
        pltpu.VMEM((H, LANES), jnp.float32),
        pltpu.SemaphoreType.DMA((2,)),
        pltpu.SemaphoreType.DMA((2,)),
        pltpu.SemaphoreType.DMA((2,)),
        pltpu.VMEM_SHARED((N, 16), jnp.float32),
    ],
)


def _p2_body(v_hbm, rsc_hbm, a_hbm, ei_hbm, z128_hbm,
             hp_hbm,
             sdidx, vrows, arows, rrows, mbuf, sem_i, sem_r, h_sh):
    c = lax.axis_index("c")
    s = lax.axis_index("s")
    wid = c * NS + s
    e0 = wid * EPW
    r0 = jnp.minimum(s * RSTEP, N - RSTEP)
    pltpu.sync_copy(z128_hbm.at[pl.ds(r0, RSTEP)], h_sh.at[pl.ds(r0, RSTEP)])
    plsc.subcore_barrier()

    iota = lax.iota(jnp.int32, LANES)

    def idx_copy(ch, b):
        base = e0 + ch * C
        return pltpu.make_async_copy(ei_hbm.at[:, pl.ds(base, C)], sdidx.at[b],
                                     sem_i.at[b])

    def row_copies(ch, b):
        base = e0 + ch * C
        return (
            pltpu.make_async_copy(v_hbm.at[sdidx.at[b, 0]], vrows.at[b], sem_r.at[b]),
            pltpu.make_async_copy(rsc_hbm.at[sdidx.at[b, 1]], rrows.at[b], sem_r.at[b]),
            pltpu.make_async_copy(a_hbm.at[pl.ds(base, C)], arows.at[b], sem_r.at[b]),
        )

    def compute(ch, b):
        vr = vrows.at[b]
        ar = arows.at[b]
        rr = rrows.at[b]

        def group(g, carry2):
            rows = g * LANES + iota
            for h in range(H):
                hcol = jnp.full((LANES,), h, jnp.int32)
                attn = (plsc.load_gather(ar, [rows, hcol])
                        * plsc.load_gather(rr, [rows, hcol]))
                for dd in range(LANES):
                    col = jnp.full((LANES,), h * LANES + dd, jnp.int32)
                    mv = plsc.load_gather(vr, [rows, col])
                    plsc.store_scatter(mbuf, [rows, col], attn * mv)
            return carry2

        lax.fori_loop(0, GRP, group, 0)
        pltpu.sync_copy(mbuf, h_sh.at[sdidx.at[b, 1]], add=True)

    idx_copy(0, 0).start()

    def step(ch, carry):
        b = lax.rem(ch, 2)
        bb = 1 - b

        @pl.when(ch < NCHUNK)
        def _fetch():
            idx_copy(ch, b).wait()
            for cp in row_copies(ch, b):
                cp.start()

        @pl.when(ch >= 1)
        def _work():
            for cp in row_copies(ch - 1, bb):
                cp.wait()
            compute(ch - 1, bb)

        @pl.when(ch + 1 < NCHUNK)
        def _prefetch_idx():
            idx_copy(ch + 1, bb).start()

        return carry

    lax.fori_loop(0, NCHUNK + 1, step, 0)
    plsc.subcore_barrier()
    pltpu.sync_copy(h_sh.at[pl.ds(r0, RSTEP)],
                    hp_hbm.at[c, pl.ds(r0, RSTEP)])


_p2 = pl.kernel(
    _p2_body,
    out_type=jax.ShapeDtypeStruct((NC, N, D), jnp.float32),
    mesh=_sc_mesh,
    compiler_params=pltpu.CompilerParams(needs_layout_passes=False, use_tc_tiling_on_sc=False),
    scratch_types=[
        pltpu.VMEM((2, 2, C), jnp.int32),
        pltpu.VMEM((2, C, D), jnp.float32),
        pltpu.VMEM((2, C, 16), jnp.float32),
        pltpu.VMEM((2, C, 16), jnp.float32),
        pltpu.VMEM((C, D), jnp.float32),
        pltpu.SemaphoreType.DMA((2,)),
        pltpu.SemaphoreType.DMA((2,)),
        pltpu.VMEM_SHARED((N, D), jnp.float32),
    ],
)


def kernel(x, edge_index, edge_attr, ck_w, ck_b, qw_w, qw_b, vw_w, vw_b,
           aw_w, aw_b, cw_w, cw_b, lx_w, lx_b, la1_w, la1_b, la2_w, la2_b):
    z16 = jnp.zeros((N, 16), jnp.float32)
    z128 = jnp.zeros((N, D), jnp.float32)

    eas = _ea_prep(edge_attr,
                   ck_w[0, :D], ck_b[0].reshape(1, D),
                   ck_w[1, :D], ck_b[1].reshape(1, D),
                   ck_w[2, :D], ck_b[2].reshape(1, D))

    for l in range(L):
        q, xk, v, lxv = _prep(x,
                              qw_w[l], qw_b[l].reshape(1, D),
                              ck_w[l, D:],
                              vw_w[l], vw_b[l].reshape(1, D),
                              lx_w[l], lx_b[l].reshape(1, D))
        awsp = jnp.broadcast_to(aw_w[l][:, :, None], (H, H, LANES))
        absp = jnp.broadcast_to(aw_b[l][:, None], (H, LANES))
        a_e, scp = _p1(q, xk, eas[l], edge_index, awsp, absp, z16)
        rsc = _combine(scp)
        hp = _p2(v, rsc, a_e, edge_index, z128)
        x = _node(x, lxv, hp, scp,
                  cw_w[l], cw_b[l].reshape(1, D),
                  la1_w[l, :D], la1_w[l, D:], la1_b[l].reshape(1, D),
                  la2_w[l, :, 0].reshape(1, D), la2_w[l, :, 1].reshape(1, D),
                  la2_b[l, 0].reshape(1, 1), la2_b[l, 1].reshape(1, 1))
    return x


# speedup vs baseline: 3.4069x; 2.8889x over previous
import functools
import math

import jax
import jax.numpy as jnp
from jax import lax
from jax.experimental import pallas as pl
from jax.experimental.pallas import tpu as pltpu
from jax.experimental.pallas import tpu_sc as plsc

N = 10000
E = 320000
D = 128
H = 8
L = 3
LANES = 16
NC = 2
NS = 16
NW = NC * NS
EPW = E // NW
C = 80
NCHUNK = EPW // C
GRP = C // LANES
RSTEP = 632

INV_SQRT_D = 1.0 / math.sqrt(D)
BN = 2000
BE = 4000


def _prep_body(x_ref, wq, bq, wk, wv, bv, wl, bl, q_ref, xk_ref, v_ref, lx_ref):
    xb = x_ref[...]
    q_ref[...] = jnp.dot(xb, wq[...], preferred_element_type=jnp.float32) + bq[...]
    xk_ref[...] = jnp.dot(xb, wk[...], preferred_element_type=jnp.float32) * INV_SQRT_D
    v_ref[...] = jnp.dot(xb, wv[...], preferred_element_type=jnp.float32) + bv[...]
    lx_ref[...] = jnp.dot(xb, wl[...], preferred_element_type=jnp.float32) + bl[...]


_prep = pl.pallas_call(
    _prep_body,
    grid=(N // BN,),
    in_specs=[
        pl.BlockSpec((BN, D), lambda i: (i, 0)),
        pl.BlockSpec((D, D), lambda i: (0, 0)),
        pl.BlockSpec((1, D), lambda i: (0, 0)),
        pl.BlockSpec((D, D), lambda i: (0, 0)),
        pl.BlockSpec((D, D), lambda i: (0, 0)),
        pl.BlockSpec((1, D), lambda i: (0, 0)),
        pl.BlockSpec((D, D), lambda i: (0, 0)),
        pl.BlockSpec((1, D), lambda i: (0, 0)),
    ],
    out_specs=[pl.BlockSpec((BN, D), lambda i: (i, 0))] * 4,
    out_shape=[jax.ShapeDtypeStruct((N, D), jnp.float32)] * 4,
)


def _ea_body(ea_ref, w0, b0, w1, b1, w2, b2, o0, o1, o2):
    eb = ea_ref[...]
    o0[...] = (jnp.dot(eb, w0[...], preferred_element_type=jnp.float32) + b0[...]) * INV_SQRT_D
    o1[...] = (jnp.dot(eb, w1[...], preferred_element_type=jnp.float32) + b1[...]) * INV_SQRT_D
    o2[...] = (jnp.dot(eb, w2[...], preferred_element_type=jnp.float32) + b2[...]) * INV_SQRT_D


_ea_prep = pl.pallas_call(
    _ea_body,
    grid=(E // BE,),
    in_specs=[pl.BlockSpec((BE, D), lambda i: (i, 0))]
    + [pl.BlockSpec((D, D), lambda i: (0, 0)), pl.BlockSpec((1, D), lambda i: (0, 0))] * 3,
    out_specs=[pl.BlockSpec((BE, D), lambda i: (i, 0))] * 3,
    out_shape=[jax.ShapeDtypeStruct((E, D), jnp.float32)] * 3,
)


def _comb_body(scp_ref, rsc_ref):
    s = scp_ref[0] + scp_ref[1]
    rsc_ref[...] = 1.0 / jnp.where(s == 0.0, 1.0, s)


_combine = pl.pallas_call(
    _comb_body,
    in_specs=[pl.BlockSpec((NC, N, 16), lambda: (0, 0, 0))],
    out_specs=pl.BlockSpec((N, 16), lambda: (0, 0)),
    out_shape=jax.ShapeDtypeStruct((N, 16), jnp.float32),
)


def _node_body(x_ref, lx_ref, hp_ref, scp_ref, cww, cwb, a1a, a1b, a1bias,
               w20, w21, b20, b21, out_ref):
    xb = x_ref[...]
    hpre = hp_ref[0] + hp_ref[1]
    deg = scp_ref[0, :, 8:9] + scp_ref[1, :, 8:9]
    h = jnp.dot(hpre, cww[...], preferred_element_type=jnp.float32) + deg * cwb[...]
    z = (jnp.dot(lx_ref[...], a1a[...], preferred_element_type=jnp.float32)
         + jnp.dot(h, a1b[...], preferred_element_type=jnp.float32) + a1bias[...])
    z = jnp.where(z > 0, z, 0.2 * z)
    p0 = jnp.sum(z * w20[...], axis=1, keepdims=True) + b20[...]
    p1 = jnp.sum(z * w21[...], axis=1, keepdims=True) + b21[...]
    m = jnp.maximum(p0, p1)
    e0 = jnp.exp(p0 - m)
    e1 = jnp.exp(p1 - m)
    inv = 1.0 / (e0 + e1)
    out_ref[...] = xb * (e0 * inv) + h * (e1 * inv)


_node = pl.pallas_call(
    _node_body,
    grid=(N // BN,),
    in_specs=[
        pl.BlockSpec((BN, D), lambda i: (i, 0)),
        pl.BlockSpec((BN, D), lambda i: (i, 0)),
        pl.BlockSpec((NC, BN, D), lambda i: (0, i, 0)),
        pl.BlockSpec((NC, BN, 16), lambda i: (0, i, 0)),
        pl.BlockSpec((D, D), lambda i: (0, 0)),
        pl.BlockSpec((1, D), lambda i: (0, 0)),
        pl.BlockSpec((D, D), lambda i: (0, 0)),
        pl.BlockSpec((D, D), lambda i: (0, 0)),
        pl.BlockSpec((1, D), lambda i: (0, 0)),
        pl.BlockSpec((1, D), lambda i: (0, 0)),
        pl.BlockSpec((1, D), lambda i: (0, 0)),
        pl.BlockSpec((1, 1), lambda i: (0, 0)),
        pl.BlockSpec((1, 1), lambda i: (0, 0)),
    ],
    out_specs=pl.BlockSpec((BN, D), lambda i: (i, 0)),
    out_shape=jax.ShapeDtypeStruct((N, D), jnp.float32),
)


_sc_mesh = plsc.VectorSubcoreMesh(core_axis_name="c", subcore_axis_name="s")


def _p1_body(q_hbm, xk_hbm, ea_hbm, ei_hbm, awsp_hbm, absp_hbm, z16_hbm,
             a_hbm, scp_hbm,
             sdidx, qrows, krows, erows, abuf, awsp_v, absp_v,
             sem_i, sem_r, sem_a, sc_sh):
    c = lax.axis_index("c")
    s = lax.axis_index("s")
    wid = c * NS + s
    e0 = wid * EPW
    r0 = jnp.minimum(s * RSTEP, N - RSTEP)
    pltpu.sync_copy(z16_hbm.at[pl.ds(r0, RSTEP)], sc_sh.at[pl.ds(r0, RSTEP)])
    pltpu.sync_copy(awsp_hbm, awsp_v)
    pltpu.sync_copy(absp_hbm, absp_v)
    plsc.subcore_barrier()

    iota = lax.iota(jnp.int32, LANES)

    def idx_copy(ch, b):
        base = e0 + ch * C
        return pltpu.make_async_copy(ei_hbm.at[:, pl.ds(base, C)], sdidx.at[b],
                                     sem_i.at[b])

    def row_copies(ch, b):
        base = e0 + ch * C
        return (
            pltpu.make_async_copy(q_hbm.at[sdidx.at[b, 1]], qrows.at[b], sem_r.at[b]),
            pltpu.make_async_copy(xk_hbm.at[sdidx.at[b, 0]], krows.at[b], sem_r.at[b]),
            pltpu.make_async_copy(ea_hbm.at[pl.ds(base, C)], erows.at[b], sem_r.at[b]),
        )

    def a_copy(ch, b):
        base = e0 + ch * C
        return pltpu.make_async_copy(abuf.at[b], a_hbm.at[pl.ds(base, C)],
                                     sem_a.at[b])

    def compute(ch, b):
        qr = qrows.at[b]
        kr = krows.at[b]
        er = erows.at[b]

        def group(g, carry2):
            rows = g * LANES + iota
            ls = []
            for h in range(H):
                acc = jnp.zeros((LANES,), jnp.float32)
                for dd in range(LANES):
                    col = h * LANES + ((dd + iota) & (LANES - 1))
                    qv = plsc.load_gather(qr, [rows, col])
                    kv = plsc.load_gather(kr, [rows, col])
                    ev = plsc.load_gather(er, [rows, col])
                    acc = acc + qv * (kv + ev)
                ls.append(jnp.where(acc > 0, acc, 0.2 * acc))
            for j in range(H):
                t = absp_v[j]
                for h in range(H):
                    t = t + ls[h] * awsp_v[h, j]
                aj = jnp.exp(t)
                plsc.store_scatter(abuf.at[b], [rows, jnp.full((LANES,), j, jnp.int32)], aj)
            plsc.store_scatter(abuf.at[b], [rows, jnp.full((LANES,), H, jnp.int32)],
                               jnp.ones((LANES,), jnp.float32))
            return carry2

        lax.fori_loop(0, GRP, group, 0)
        a_copy(ch, b).start()
        pltpu.sync_copy(abuf.at[b], sc_sh.at[sdidx.at[b, 1]], add=True)

    idx_copy(0, 0).start()

    def step(ch, carry):
        b = lax.rem(ch, 2)
        bb = 1 - b

        @pl.when(ch < NCHUNK)
        def _fetch():
            idx_copy(ch, b).wait()
            for cp in row_copies(ch, b):
                cp.start()

        @pl.when(ch >= 1)
        def _work():
            for cp in row_copies(ch - 1, bb):
                cp.wait()

            @pl.when(ch >= 3)
            def _drain_a():
                a_copy(ch - 3, bb).wait()

            compute(ch - 1, bb)

        @pl.when(ch + 1 < NCHUNK)
        def _prefetch_idx():
            idx_copy(ch + 1, bb).start()

        return carry

    lax.fori_loop(0, NCHUNK + 1, step, 0)
    a_copy(NCHUNK - 2, lax.rem(NCHUNK - 2, 2)).wait()
    a_copy(NCHUNK - 1, lax.rem(NCHUNK - 1, 2)).wait()
    plsc.subcore_barrier()
    pltpu.sync_copy(sc_sh.at[pl.ds(r0, RSTEP)],
                    scp_hbm.at[c, pl.ds(r0, RSTEP)])


_p1 = pl.kernel(
    _p1_body,
    out_type=[jax.ShapeDtypeStruct((E, 16), jnp.float32),
              jax.ShapeDtypeStruct((NC, N, 16), jnp.float32)],
    mesh=_sc_mesh,
    compiler_params=pltpu.CompilerParams(needs_layout_passes=False, use_tc_tiling_on_sc=False),
    scratch_types=[
        pltpu.VMEM((2, 2, C), jnp.int32),
        pltpu.VMEM((2, C, D), jnp.float32),
        pltpu.VMEM((2, C, D), jnp.float32),
        pltpu.VMEM((2, C, D), jnp.float32),
        pltpu.VMEM((2, C, 16), jnp.float32),
        pltpu.VMEM((H, H, LANES), jnp.float32),
        pltpu.VMEM((H, LANES), jnp.float32),
        pltpu.SemaphoreType.DMA((2,)),
        pltpu.SemaphoreType.DMA((2,)),
        pltpu.SemaphoreType.DMA((2,)),
        pltpu.VMEM_SHARED((N, 16), jnp.float32),
    ],
)


def _p2_body(v_hbm, rsc_hbm, a_hbm, ei_hbm, z128_hbm,
             hp_hbm,
             sdidx, vrows, arows, rrows, mbuf, sem_i, sem_r, h_sh):
    c = lax.axis_index("c")
    s = lax.axis_index("s")
    wid = c * NS + s
    e0 = wid * EPW
    r0 = jnp.minimum(s * RSTEP, N - RSTEP)
    pltpu.sync_copy(z128_hbm.at[pl.ds(r0, RSTEP)], h_sh.at[pl.ds(r0, RSTEP)])
    plsc.subcore_barrier()

    iota = lax.iota(jnp.int32, LANES)

    def idx_copy(ch, b):
        base = e0 + ch * C
        return pltpu.make_async_copy(ei_hbm.at[:, pl.ds(base, C)], sdidx.at[b],
                                     sem_i.at[b])

    def row_copies(ch, b):
        base = e0 + ch * C
        return (
            pltpu.make_async_copy(v_hbm.at[sdidx.at[b, 0]], vrows.at[b], sem_r.at[b]),
            pltpu.make_async_copy(rsc_hbm.at[sdidx.at[b, 1]], rrows.at[b], sem_r.at[b]),
            pltpu.make_async_copy(a_hbm.at[pl.ds(base, C)], arows.at[b], sem_r.at[b]),
        )

    def compute(ch, b):
        vr = vrows.at[b]
        ar = arows.at[b]
        rr = rrows.at[b]

        def group(g, carry2):
            rows = g * LANES + iota
            for h in range(H):
                hcol = jnp.full((LANES,), h, jnp.int32)
                attn = (plsc.load_gather(ar, [rows, hcol])
                        * plsc.load_gather(rr, [rows, hcol]))
                for dd in range(LANES):
                    col = h * LANES + ((dd + iota) & (LANES - 1))
                    mv = plsc.load_gather(vr, [rows, col])
                    plsc.store_scatter(mbuf, [rows, col], attn * mv)
            return carry2

        lax.fori_loop(0, GRP, group, 0)
        pltpu.sync_copy(mbuf, h_sh.at[sdidx.at[b, 1]], add=True)

    idx_copy(0, 0).start()

    def step(ch, carry):
        b = lax.rem(ch, 2)
        bb = 1 - b

        @pl.when(ch < NCHUNK)
        def _fetch():
            idx_copy(ch, b).wait()
            for cp in row_copies(ch, b):
                cp.start()

        @pl.when(ch >= 1)
        def _work():
            for cp in row_copies(ch - 1, bb):
                cp.wait()
            compute(ch - 1, bb)

        @pl.when(ch + 1 < NCHUNK)
        def _prefetch_idx():
            idx_copy(ch + 1, bb).start()

        return carry

    lax.fori_loop(0, NCHUNK + 1, step, 0)
    plsc.subcore_barrier()
    pltpu.sync_copy(h_sh.at[pl.ds(r0, RSTEP)],
                    hp_hbm.at[c, pl.ds(r0, RSTEP)])


_p2 = pl.kernel(
    _p2_body,
    out_type=jax.ShapeDtypeStruct((NC, N, D), jnp.float32),
    mesh=_sc_mesh,
    compiler_params=pltpu.CompilerParams(needs_layout_passes=False, use_tc_tiling_on_sc=False),
    scratch_types=[
        pltpu.VMEM((2, 2, C), jnp.int32),
        pltpu.VMEM((2, C, D), jnp.float32),
        pltpu.VMEM((2, C, 16), jnp.float32),
        pltpu.VMEM((2, C, 16), jnp.float32),
        pltpu.VMEM((C, D), jnp.float32),
        pltpu.SemaphoreType.DMA((2,)),
        pltpu.SemaphoreType.DMA((2,)),
        pltpu.VMEM_SHARED((N, D), jnp.float32),
    ],
)


def kernel(x, edge_index, edge_attr, ck_w, ck_b, qw_w, qw_b, vw_w, vw_b,
           aw_w, aw_b, cw_w, cw_b, lx_w, lx_b, la1_w, la1_b, la2_w, la2_b):
    z16 = jnp.zeros((N, 16), jnp.float32)
    z128 = jnp.zeros((N, D), jnp.float32)

    eas = _ea_prep(edge_attr,
                   ck_w[0, :D], ck_b[0].reshape(1, D),
                   ck_w[1, :D], ck_b[1].reshape(1, D),
                   ck_w[2, :D], ck_b[2].reshape(1, D))

    for l in range(L):
        q, xk, v, lxv = _prep(x,
                              qw_w[l], qw_b[l].reshape(1, D),
                              ck_w[l, D:],
                              vw_w[l], vw_b[l].reshape(1, D),
                              lx_w[l], lx_b[l].reshape(1, D))
        awsp = jnp.broadcast_to(aw_w[l][:, :, None], (H, H, LANES))
        absp = jnp.broadcast_to(aw_b[l][:, None], (H, LANES))
        a_e, scp = _p1(q, xk, eas[l], edge_index, awsp, absp, z16)
        rsc = _combine(scp)
        hp = _p2(v, rsc, a_e, edge_index, z128)
        x = _node(x, lxv, hp, scp,
                  cw_w[l], cw_b[l].reshape(1, D),
                  la1_w[l, :D], la1_w[l, D:], la1_b[l].reshape(1, D),
                  la2_w[l, :, 0].reshape(1, D), la2_w[l, :, 1].reshape(1, D),
                  la2_b[l, 0].reshape(1, 1), la2_b[l, 1].reshape(1, 1))
    return x


# speedup vs baseline: 3.4833x; 1.0224x over previous
import functools
import math

import jax
import jax.numpy as jnp
from jax import lax
from jax.experimental import pallas as pl
from jax.experimental.pallas import tpu as pltpu
from jax.experimental.pallas import tpu_sc as plsc

N = 10000
E = 320000
D = 128
H = 8
L = 3
LANES = 16
NC = 2
NS = 16
NW = NC * NS
EPW = E // NW
C = 80
NCHUNK = EPW // C
GRP = C // LANES
RSTEP = 632

INV_SQRT_D = 1.0 / math.sqrt(D)
BN = 2000
BE = 4000


def _prep_body(x_ref, wq, bq, wk, wv, bv, wl, bl, q_ref, xk_ref, v_ref, lx_ref):
    xb = x_ref[...]
    q_ref[...] = jnp.dot(xb, wq[...], preferred_element_type=jnp.float32) + bq[...]
    xk_ref[...] = jnp.dot(xb, wk[...], preferred_element_type=jnp.float32) * INV_SQRT_D
    v_ref[...] = jnp.dot(xb, wv[...], preferred_element_type=jnp.float32) + bv[...]
    lx_ref[...] = jnp.dot(xb, wl[...], preferred_element_type=jnp.float32) + bl[...]


_prep = pl.pallas_call(
    _prep_body,
    grid=(N // BN,),
    in_specs=[
        pl.BlockSpec((BN, D), lambda i: (i, 0)),
        pl.BlockSpec((D, D), lambda i: (0, 0)),
        pl.BlockSpec((1, D), lambda i: (0, 0)),
        pl.BlockSpec((D, D), lambda i: (0, 0)),
        pl.BlockSpec((D, D), lambda i: (0, 0)),
        pl.BlockSpec((1, D), lambda i: (0, 0)),
        pl.BlockSpec((D, D), lambda i: (0, 0)),
        pl.BlockSpec((1, D), lambda i: (0, 0)),
    ],
    out_specs=[pl.BlockSpec((BN, D), lambda i: (i, 0))] * 4,
    out_shape=[jax.ShapeDtypeStruct((N, D), jnp.float32)] * 4,
)


def _ea_body(ea_ref, w0, b0, w1, b1, w2, b2, o0, o1, o2):
    eb = ea_ref[...]
    o0[...] = (jnp.dot(eb, w0[...], preferred_element_type=jnp.float32) + b0[...]) * INV_SQRT_D
    o1[...] = (jnp.dot(eb, w1[...], preferred_element_type=jnp.float32) + b1[...]) * INV_SQRT_D
    o2[...] = (jnp.dot(eb, w2[...], preferred_element_type=jnp.float32) + b2[...]) * INV_SQRT_D


_ea_prep = pl.pallas_call(
    _ea_body,
    grid=(E // BE,),
    in_specs=[pl.BlockSpec((BE, D), lambda i: (i, 0))]
    + [pl.BlockSpec((D, D), lambda i: (0, 0)), pl.BlockSpec((1, D), lambda i: (0, 0))] * 3,
    out_specs=[pl.BlockSpec((BE, D), lambda i: (i, 0))] * 3,
    out_shape=[jax.ShapeDtypeStruct((E, D), jnp.float32)] * 3,
)


def _node_body(x_ref, lx_ref, hp_ref, scp_ref, cww, cwb, a1a, a1b, a1bias,
               w20, w21, b20, b21, out_ref):
    xb = x_ref[...]
    hraw = hp_ref[0] + hp_ref[1]
    sc = scp_ref[0] + scp_ref[1]
    deg = sc[:, 8:9]
    inv_sc = 1.0 / jnp.where(sc == 0.0, 1.0, sc)
    hpre = jnp.concatenate(
        [hraw[:, hh * LANES:(hh + 1) * LANES] * inv_sc[:, hh:hh + 1]
         for hh in range(H)], axis=1)
    h = jnp.dot(hpre, cww[...], preferred_element_type=jnp.float32) + deg * cwb[...]
    z = (jnp.dot(lx_ref[...], a1a[...], preferred_element_type=jnp.float32)
         + jnp.dot(h, a1b[...], preferred_element_type=jnp.float32) + a1bias[...])
    z = jnp.where(z > 0, z, 0.2 * z)
    p0 = jnp.sum(z * w20[...], axis=1, keepdims=True) + b20[...]
    p1 = jnp.sum(z * w21[...], axis=1, keepdims=True) + b21[...]
    m = jnp.maximum(p0, p1)
    e0 = jnp.exp(p0 - m)
    e1 = jnp.exp(p1 - m)
    inv = 1.0 / (e0 + e1)
    out_ref[...] = xb * (e0 * inv) + h * (e1 * inv)


_node = pl.pallas_call(
    _node_body,
    grid=(N // BN,),
    in_specs=[
        pl.BlockSpec((BN, D), lambda i: (i, 0)),
        pl.BlockSpec((BN, D), lambda i: (i, 0)),
        pl.BlockSpec((NC, BN, D), lambda i: (0, i, 0)),
        pl.BlockSpec((NC, BN, 16), lambda i: (0, i, 0)),
        pl.BlockSpec((D, D), lambda i: (0, 0)),
        pl.BlockSpec((1, D), lambda i: (0, 0)),
        pl.BlockSpec((D, D), lambda i: (0, 0)),
        pl.BlockSpec((D, D), lambda i: (0, 0)),
        pl.BlockSpec((1, D), lambda i: (0, 0)),
        pl.BlockSpec((1, D), lambda i: (0, 0)),
        pl.BlockSpec((1, D), lambda i: (0, 0)),
        pl.BlockSpec((1, 1), lambda i: (0, 0)),
        pl.BlockSpec((1, 1), lambda i: (0, 0)),
    ],
    out_specs=pl.BlockSpec((BN, D), lambda i: (i, 0)),
    out_shape=jax.ShapeDtypeStruct((N, D), jnp.float32),
)


_sc_mesh = plsc.VectorSubcoreMesh(core_axis_name="c", subcore_axis_name="s")


def _p1_body(q_hbm, xk_hbm, ea_hbm, ei_hbm, awsp_hbm, absp_hbm, z16_hbm,
             a_hbm, scp_hbm,
             sdidx, qrows, krows, erows, abuf, awsp_v, absp_v,
             sem_i, sem_r, sem_a, sc_sh):
    c = lax.axis_index("c")
    s = lax.axis_index("s")
    wid = c * NS + s
    e0 = wid * EPW
    r0 = jnp.minimum(s * RSTEP, N - RSTEP)
    pltpu.sync_copy(z16_hbm.at[pl.ds(r0, RSTEP)], sc_sh.at[pl.ds(r0, RSTEP)])
    pltpu.sync_copy(awsp_hbm, awsp_v)
    pltpu.sync_copy(absp_hbm, absp_v)
    plsc.subcore_barrier()

    iota = lax.iota(jnp.int32, LANES)

    def idx_copy(ch, b):
        base = e0 + ch * C
        return pltpu.make_async_copy(ei_hbm.at[:, pl.ds(base, C)], sdidx.at[b],
                                     sem_i.at[b])

    def row_copies(ch, b):
        base = e0 + ch * C
        return (
            pltpu.make_async_copy(q_hbm.at[sdidx.at[b, 1]], qrows.at[b], sem_r.at[b]),
            pltpu.make_async_copy(xk_hbm.at[sdidx.at[b, 0]], krows.at[b], sem_r.at[b]),
            pltpu.make_async_copy(ea_hbm.at[pl.ds(base, C)], erows.at[b], sem_r.at[b]),
        )

    def a_copy(ch, b):
        base = e0 + ch * C
        return pltpu.make_async_copy(abuf.at[b], a_hbm.at[pl.ds(base, C)],
                                     sem_a.at[b])

    def compute(ch, b):
        qr = qrows.at[b]
        kr = krows.at[b]
        er = erows.at[b]

        def group(g, carry2):
            rows = g * LANES + iota
            ls = []
            for h in range(H):
                acc = jnp.zeros((LANES,), jnp.float32)
                for dd in range(LANES):
                    col = h * LANES + ((dd + iota) & (LANES - 1))
                    qv = plsc.load_gather(qr, [rows, col])
                    kv = plsc.load_gather(kr, [rows, col])
                    ev = plsc.load_gather(er, [rows, col])
                    acc = acc + qv * (kv + ev)
                ls.append(jnp.where(acc > 0, acc, 0.2 * acc))
            for j in range(H):
                t = absp_v[j]
                for h in range(H):
                    t = t + ls[h] * awsp_v[h, j]
                aj = jnp.exp(t)
                plsc.store_scatter(abuf.at[b], [rows, jnp.full((LANES,), j, jnp.int32)], aj)
            plsc.store_scatter(abuf.at[b], [rows, jnp.full((LANES,), H, jnp.int32)],
                               jnp.ones((LANES,), jnp.float32))
            return carry2

        lax.fori_loop(0, GRP, group, 0)
        a_copy(ch, b).start()
        pltpu.sync_copy(abuf.at[b], sc_sh.at[sdidx.at[b, 1]], add=True)

    idx_copy(0, 0).start()

    def step(ch, carry):
        b = lax.rem(ch, 2)
        bb = 1 - b

        @pl.when(ch < NCHUNK)
        def _fetch():
            idx_copy(ch, b).wait()
            for cp in row_copies(ch, b):
                cp.start()

        @pl.when(ch >= 1)
        def _work():
            for cp in row_copies(ch - 1, bb):
                cp.wait()

            @pl.when(ch >= 3)
            def _drain_a():
                a_copy(ch - 3, bb).wait()

            compute(ch - 1, bb)

        @pl.when(ch + 1 < NCHUNK)
        def _prefetch_idx():
            idx_copy(ch + 1, bb).start()

        return carry

    lax.fori_loop(0, NCHUNK + 1, step, 0)
    a_copy(NCHUNK - 2, lax.rem(NCHUNK - 2, 2)).wait()
    a_copy(NCHUNK - 1, lax.rem(NCHUNK - 1, 2)).wait()
    plsc.subcore_barrier()
    pltpu.sync_copy(sc_sh.at[pl.ds(r0, RSTEP)],
                    scp_hbm.at[c, pl.ds(r0, RSTEP)])


_p1 = pl.kernel(
    _p1_body,
    out_type=[jax.ShapeDtypeStruct((E, 16), jnp.float32),
              jax.ShapeDtypeStruct((NC, N, 16), jnp.float32)],
    mesh=_sc_mesh,
    compiler_params=pltpu.CompilerParams(needs_layout_passes=False, use_tc_tiling_on_sc=False),
    scratch_types=[
        pltpu.VMEM((2, 2, C), jnp.int32),
        pltpu.VMEM((2, C, D), jnp.float32),
        pltpu.VMEM((2, C, D), jnp.float32),
        pltpu.VMEM((2, C, D), jnp.float32),
        pltpu.VMEM((2, C, 16), jnp.float32),
        pltpu.VMEM((H, H, LANES), jnp.float32),
        pltpu.VMEM((H, LANES), jnp.float32),
        pltpu.SemaphoreType.DMA((2,)),
        pltpu.SemaphoreType.DMA((2,)),
        pltpu.SemaphoreType.DMA((2,)),
        pltpu.VMEM_SHARED((N, 16), jnp.float32),
    ],
)


def _p2_body(v_hbm, a_hbm, ei_hbm, z128_hbm,
             hp_hbm,
             sdidx, vrows, arows, mbuf, sem_i, sem_r, h_sh):
    c = lax.axis_index("c")
    s = lax.axis_index("s")
    wid = c * NS + s
    e0 = wid * EPW
    r0 = jnp.minimum(s * RSTEP, N - RSTEP)
    pltpu.sync_copy(z128_hbm.at[pl.ds(r0, RSTEP)], h_sh.at[pl.ds(r0, RSTEP)])
    plsc.subcore_barrier()

    iota = lax.iota(jnp.int32, LANES)

    def idx_copy(ch, b):
        base = e0 + ch * C
        return pltpu.make_async_copy(ei_hbm.at[:, pl.ds(base, C)], sdidx.at[b],
                                     sem_i.at[b])

    def row_copies(ch, b):
        base = e0 + ch * C
        return (
            pltpu.make_async_copy(v_hbm.at[sdidx.at[b, 0]], vrows.at[b], sem_r.at[b]),
            pltpu.make_async_copy(a_hbm.at[pl.ds(base, C)], arows.at[b], sem_r.at[b]),
        )

    def compute(ch, b):
        vr = vrows.at[b]
        ar = arows.at[b]

        def group(g, carry2):
            rows = g * LANES + iota
            for h in range(H):
                hcol = jnp.full((LANES,), h, jnp.int32)
                attn = plsc.load_gather(ar, [rows, hcol])
                for dd in range(LANES):
                    col = h * LANES + ((dd + iota) & (LANES - 1))
                    mv = plsc.load_gather(vr, [rows, col])
                    plsc.store_scatter(mbuf, [rows, col], attn * mv)
            return carry2

        lax.fori_loop(0, GRP, group, 0)
        pltpu.sync_copy(mbuf, h_sh.at[sdidx.at[b, 1]], add=True)

    idx_copy(0, 0).start()

    def step(ch, carry):
        b = lax.rem(ch, 2)
        bb = 1 - b

        @pl.when(ch < NCHUNK)
        def _fetch():
            idx_copy(ch, b).wait()
            for cp in row_copies(ch, b):
                cp.start()

        @pl.when(ch >= 1)
        def _work():
            for cp in row_copies(ch - 1, bb):
                cp.wait()
            compute(ch - 1, bb)

        @pl.when(ch + 1 < NCHUNK)
        def _prefetch_idx():
            idx_copy(ch + 1, bb).start()

        return carry

    lax.fori_loop(0, NCHUNK + 1, step, 0)
    plsc.subcore_barrier()
    pltpu.sync_copy(h_sh.at[pl.ds(r0, RSTEP)],
                    hp_hbm.at[c, pl.ds(r0, RSTEP)])


_p2 = pl.kernel(
    _p2_body,
    out_type=jax.ShapeDtypeStruct((NC, N, D), jnp.float32),
    mesh=_sc_mesh,
    compiler_params=pltpu.CompilerParams(needs_layout_passes=False, use_tc_tiling_on_sc=False),
    scratch_types=[
        pltpu.VMEM((2, 2, C), jnp.int32),
        pltpu.VMEM((2, C, D), jnp.float32),
        pltpu.VMEM((2, C, 16), jnp.float32),
        pltpu.VMEM((C, D), jnp.float32),
        pltpu.SemaphoreType.DMA((2,)),
        pltpu.SemaphoreType.DMA((2,)),
        pltpu.VMEM_SHARED((N, D), jnp.float32),
    ],
)


def kernel(x, edge_index, edge_attr, ck_w, ck_b, qw_w, qw_b, vw_w, vw_b,
           aw_w, aw_b, cw_w, cw_b, lx_w, lx_b, la1_w, la1_b, la2_w, la2_b):
    z16 = jnp.zeros((N, 16), jnp.float32)
    z128 = jnp.zeros((N, D), jnp.float32)

    eas = _ea_prep(edge_attr,
                   ck_w[0, :D], ck_b[0].reshape(1, D),
                   ck_w[1, :D], ck_b[1].reshape(1, D),
                   ck_w[2, :D], ck_b[2].reshape(1, D))

    for l in range(L):
        q, xk, v, lxv = _prep(x,
                              qw_w[l], qw_b[l].reshape(1, D),
                              ck_w[l, D:],
                              vw_w[l], vw_b[l].reshape(1, D),
                              lx_w[l], lx_b[l].reshape(1, D))
        awsp = jnp.broadcast_to(aw_w[l][:, :, None], (H, H, LANES))
        absp = jnp.broadcast_to(aw_b[l][:, None], (H, LANES))
        a_e, scp = _p1(q, xk, eas[l], edge_index, awsp, absp, z16)
        hp = _p2(v, a_e, edge_index, z128)
        x = _node(x, lxv, hp, scp,
                  cw_w[l], cw_b[l].reshape(1, D),
                  la1_w[l, :D], la1_w[l, D:], la1_b[l].reshape(1, D),
                  la2_w[l, :, 0].reshape(1, D), la2_w[l, :, 1].reshape(1, D),
                  la2_b[l, 0].reshape(1, 1), la2_b[l, 1].reshape(1, 1))
    return x


# speedup vs baseline: 3.6456x; 1.0466x over previous
import functools
import math

import jax
import jax.numpy as jnp
from jax import lax
from jax.experimental import pallas as pl
from jax.experimental.pallas import tpu as pltpu
from jax.experimental.pallas import tpu_sc as plsc

N = 10000
E = 320000
D = 128
H = 8
L = 3
LANES = 16
NC = 2
NS = 16
NW = NC * NS
EPW = E // NW
C = 80
NCHUNK = EPW // C
GRP = C // LANES
RSTEP = 632

INV_SQRT_D = 1.0 / math.sqrt(D)
BN = 2000
BE = 4000


def _prep_body(x_ref, wq, bq, wk, wv, bv, wl, bl, q_ref, xk_ref, v_ref, lx_ref):
    xb = x_ref[...]
    q_ref[...] = jnp.dot(xb, wq[...], preferred_element_type=jnp.float32) + bq[...]
    xk_ref[...] = jnp.dot(xb, wk[...], preferred_element_type=jnp.float32) * INV_SQRT_D
    v_ref[...] = jnp.dot(xb, wv[...], preferred_element_type=jnp.float32) + bv[...]
    lx_ref[...] = jnp.dot(xb, wl[...], preferred_element_type=jnp.float32) + bl[...]


_prep = pl.pallas_call(
    _prep_body,
    grid=(N // BN,),
    in_specs=[
        pl.BlockSpec((BN, D), lambda i: (i, 0)),
        pl.BlockSpec((D, D), lambda i: (0, 0)),
        pl.BlockSpec((1, D), lambda i: (0, 0)),
        pl.BlockSpec((D, D), lambda i: (0, 0)),
        pl.BlockSpec((D, D), lambda i: (0, 0)),
        pl.BlockSpec((1, D), lambda i: (0, 0)),
        pl.BlockSpec((D, D), lambda i: (0, 0)),
        pl.BlockSpec((1, D), lambda i: (0, 0)),
    ],
    out_specs=[pl.BlockSpec((BN, D), lambda i: (i, 0))] * 4,
    out_shape=[jax.ShapeDtypeStruct((N, D), jnp.float32)] * 4,
)


def _ea_body(ea_ref, w0, b0, w1, b1, w2, b2, o0, o1, o2):
    eb = ea_ref[...]
    o0[...] = (jnp.dot(eb, w0[...], preferred_element_type=jnp.float32) + b0[...]) * INV_SQRT_D
    o1[...] = (jnp.dot(eb, w1[...], preferred_element_type=jnp.float32) + b1[...]) * INV_SQRT_D
    o2[...] = (jnp.dot(eb, w2[...], preferred_element_type=jnp.float32) + b2[...]) * INV_SQRT_D


_ea_prep = pl.pallas_call(
    _ea_body,
    grid=(E // BE,),
    in_specs=[pl.BlockSpec((BE, D), lambda i: (i, 0))]
    + [pl.BlockSpec((D, D), lambda i: (0, 0)), pl.BlockSpec((1, D), lambda i: (0, 0))] * 3,
    out_specs=[pl.BlockSpec((BE, D), lambda i: (i, 0))] * 3,
    out_shape=[jax.ShapeDtypeStruct((E, D), jnp.float32)] * 3,
)


def _node_body(x_ref, lx_ref, hp_ref, scp_ref, cww, cwb, a1a, a1b, a1bias,
               w20, w21, b20, b21, out_ref):
    xb = x_ref[...]
    hraw = hp_ref[0] + hp_ref[1]
    sc = scp_ref[0] + scp_ref[1]
    deg = sc[:, 8:9]
    inv_sc = 1.0 / jnp.where(sc == 0.0, 1.0, sc)
    hpre = jnp.concatenate(
        [hraw[:, hh * LANES:(hh + 1) * LANES] * inv_sc[:, hh:hh + 1]
         for hh in range(H)], axis=1)
    h = jnp.dot(hpre, cww[...], preferred_element_type=jnp.float32) + deg * cwb[...]
    z = (jnp.dot(lx_ref[...], a1a[...], preferred_element_type=jnp.float32)
         + jnp.dot(h, a1b[...], preferred_element_type=jnp.float32) + a1bias[...])
    z = jnp.where(z > 0, z, 0.2 * z)
    p0 = jnp.sum(z * w20[...], axis=1, keepdims=True) + b20[...]
    p1 = jnp.sum(z * w21[...], axis=1, keepdims=True) + b21[...]
    m = jnp.maximum(p0, p1)
    e0 = jnp.exp(p0 - m)
    e1 = jnp.exp(p1 - m)
    inv = 1.0 / (e0 + e1)
    out_ref[...] = xb * (e0 * inv) + h * (e1 * inv)


_node = pl.pallas_call(
    _node_body,
    grid=(N // BN,),
    in_specs=[
        pl.BlockSpec((BN, D), lambda i: (i, 0)),
        pl.BlockSpec((BN, D), lambda i: (i, 0)),
        pl.BlockSpec((NC, BN, D), lambda i: (0, i, 0)),
        pl.BlockSpec((NC, BN, 16), lambda i: (0, i, 0)),
        pl.BlockSpec((D, D), lambda i: (0, 0)),
        pl.BlockSpec((1, D), lambda i: (0, 0)),
        pl.BlockSpec((D, D), lambda i: (0, 0)),
        pl.BlockSpec((D, D), lambda i: (0, 0)),
        pl.BlockSpec((1, D), lambda i: (0, 0)),
        pl.BlockSpec((1, D), lambda i: (0, 0)),
        pl.BlockSpec((1, D), lambda i: (0, 0)),
        pl.BlockSpec((1, 1), lambda i: (0, 0)),
        pl.BlockSpec((1, 1), lambda i: (0, 0)),
    ],
    out_specs=pl.BlockSpec((BN, D), lambda i: (i, 0)),
    out_shape=jax.ShapeDtypeStruct((N, D), jnp.float32),
)


_sc_mesh = plsc.VectorSubcoreMesh(core_axis_name="c", subcore_axis_name="s")


def _p1_body(q_hbm, xk_hbm, ea_hbm, ei_hbm, awsp_hbm, absp_hbm, z16_hbm,
             a_hbm, scp_hbm,
             sdidx, qrows, krows, erows, abuf, awsp_v, absp_v,
             sem_i, sem_r, sem_a, sem_s, sc_sh):
    c = lax.axis_index("c")
    s = lax.axis_index("s")
    wid = c * NS + s
    e0 = wid * EPW
    r0 = jnp.minimum(s * RSTEP, N - RSTEP)
    pltpu.sync_copy(z16_hbm.at[pl.ds(r0, RSTEP)], sc_sh.at[pl.ds(r0, RSTEP)])
    pltpu.sync_copy(awsp_hbm, awsp_v)
    pltpu.sync_copy(absp_hbm, absp_v)
    plsc.subcore_barrier()

    iota = lax.iota(jnp.int32, LANES)

    def idx_copy(ch):
        bi = lax.rem(ch, 4)
        base = e0 + ch * C
        return pltpu.make_async_copy(ei_hbm.at[:, pl.ds(base, C)], sdidx.at[bi],
                                     sem_i.at[lax.rem(ch, 2)])

    def row_copies(ch, b):
        bi = lax.rem(ch, 4)
        base = e0 + ch * C
        return (
            pltpu.make_async_copy(q_hbm.at[sdidx.at[bi, 1]], qrows.at[b], sem_r.at[b]),
            pltpu.make_async_copy(xk_hbm.at[sdidx.at[bi, 0]], krows.at[b], sem_r.at[b]),
            pltpu.make_async_copy(ea_hbm.at[pl.ds(base, C)], erows.at[b], sem_r.at[b]),
        )

    def a_copy(ch, b):
        base = e0 + ch * C
        return pltpu.make_async_copy(abuf.at[b], a_hbm.at[pl.ds(base, C)],
                                     sem_a.at[b])

    def sc_scatter(ch, b):
        bi = lax.rem(ch, 4)
        return pltpu.make_async_copy(abuf.at[b], sc_sh.at[sdidx.at[bi, 1]],
                                     sem_s.at[b])

    def compute(ch, b):
        qr = qrows.at[b]
        kr = krows.at[b]
        er = erows.at[b]

        def group(g, carry2):
            rows = g * LANES + iota
            ls = []
            for h in range(H):
                acc = jnp.zeros((LANES,), jnp.float32)
                for dd in range(LANES):
                    col = h * LANES + ((dd + iota) & (LANES - 1))
                    qv = plsc.load_gather(qr, [rows, col])
                    kv = plsc.load_gather(kr, [rows, col])
                    ev = plsc.load_gather(er, [rows, col])
                    acc = acc + qv * (kv + ev)
                ls.append(jnp.where(acc > 0, acc, 0.2 * acc))
            for j in range(H):
                t = absp_v[j]
                for h in range(H):
                    t = t + ls[h] * awsp_v[h, j]
                aj = jnp.exp(t)
                plsc.store_scatter(abuf.at[b], [rows, jnp.full((LANES,), j, jnp.int32)], aj)
            plsc.store_scatter(abuf.at[b], [rows, jnp.full((LANES,), H, jnp.int32)],
                               jnp.ones((LANES,), jnp.float32))
            return carry2

        lax.fori_loop(0, GRP, group, 0)
        a_copy(ch, b).start()
        sc_scatter(ch, b).start(add=True)

    idx_copy(0).start()

    def step(ch, carry):
        b = lax.rem(ch, 2)
        bb = 1 - b

        @pl.when(ch < NCHUNK)
        def _fetch():
            idx_copy(ch).wait()
            for cp in row_copies(ch, b):
                cp.start()

        @pl.when(ch >= 1)
        def _work():
            for cp in row_copies(ch - 1, bb):
                cp.wait()

            @pl.when(ch >= 3)
            def _drain_prev():
                a_copy(ch - 3, bb).wait()
                sc_scatter(ch - 3, bb).wait()

            compute(ch - 1, bb)

        @pl.when(ch + 1 < NCHUNK)
        def _prefetch_idx():
            idx_copy(ch + 1).start()

        return carry

    lax.fori_loop(0, NCHUNK + 1, step, 0)
    for cc in (NCHUNK - 2, NCHUNK - 1):
        a_copy(cc, cc % 2).wait()
        sc_scatter(cc, cc % 2).wait()
    plsc.subcore_barrier()
    pltpu.sync_copy(sc_sh.at[pl.ds(r0, RSTEP)],
                    scp_hbm.at[c, pl.ds(r0, RSTEP)])


_p1 = pl.kernel(
    _p1_body,
    out_type=[jax.ShapeDtypeStruct((E, 16), jnp.float32),
              jax.ShapeDtypeStruct((NC, N, 16), jnp.float32)],
    mesh=_sc_mesh,
    compiler_params=pltpu.CompilerParams(needs_layout_passes=False, use_tc_tiling_on_sc=False),
    scratch_types=[
        pltpu.VMEM((4, 2, C), jnp.int32),
        pltpu.VMEM((2, C, D), jnp.float32),
        pltpu.VMEM((2, C, D), jnp.float32),
        pltpu.VMEM((2, C, D), jnp.float32),
        pltpu.VMEM((2, C, 16), jnp.float32),
        pltpu.VMEM((H, H, LANES), jnp.float32),
        pltpu.VMEM((H, LANES), jnp.float32),
        pltpu.SemaphoreType.DMA((2,)),
        pltpu.SemaphoreType.DMA((2,)),
        pltpu.SemaphoreType.DMA((2,)),
        pltpu.SemaphoreType.DMA((2,)),
        pltpu.VMEM_SHARED((N, 16), jnp.float32),
    ],
)


def _p2_body(v_hbm, a_hbm, ei_hbm, z128_hbm,
             hp_hbm,
             sdidx, vrows, arows, mbuf, sem_i, sem_r, sem_s, h_sh):
    c = lax.axis_index("c")
    s = lax.axis_index("s")
    wid = c * NS + s
    e0 = wid * EPW
    r0 = jnp.minimum(s * RSTEP, N - RSTEP)
    pltpu.sync_copy(z128_hbm.at[pl.ds(r0, RSTEP)], h_sh.at[pl.ds(r0, RSTEP)])
    plsc.subcore_barrier()

    iota = lax.iota(jnp.int32, LANES)

    def idx_copy(ch):
        bi = lax.rem(ch, 4)
        base = e0 + ch * C
        return pltpu.make_async_copy(ei_hbm.at[:, pl.ds(base, C)], sdidx.at[bi],
                                     sem_i.at[lax.rem(ch, 2)])

    def row_copies(ch, b):
        bi = lax.rem(ch, 4)
        base = e0 + ch * C
        return (
            pltpu.make_async_copy(v_hbm.at[sdidx.at[bi, 0]], vrows.at[b], sem_r.at[b]),
            pltpu.make_async_copy(a_hbm.at[pl.ds(base, C)], arows.at[b], sem_r.at[b]),
        )

    def m_scatter(ch, b):
        bi = lax.rem(ch, 4)
        return pltpu.make_async_copy(mbuf.at[b], h_sh.at[sdidx.at[bi, 1]],
                                     sem_s.at[b])

    def compute(ch, b):
        vr = vrows.at[b]
        ar = arows.at[b]

        def group(g, carry2):
            rows = g * LANES + iota
            for h in range(H):
                hcol = jnp.full((LANES,), h, jnp.int32)
                attn = plsc.load_gather(ar, [rows, hcol])
                for dd in range(LANES):
                    col = h * LANES + ((dd + iota) & (LANES - 1))
                    mv = plsc.load_gather(vr, [rows, col])
                    plsc.store_scatter(mbuf.at[b], [rows, col], attn * mv)
            return carry2

        lax.fori_loop(0, GRP, group, 0)
        m_scatter(ch, b).start(add=True)

    idx_copy(0).start()

    def step(ch, carry):
        b = lax.rem(ch, 2)
        bb = 1 - b

        @pl.when(ch < NCHUNK)
        def _fetch():
            idx_copy(ch).wait()
            for cp in row_copies(ch, b):
                cp.start()

        @pl.when(ch >= 1)
        def _work():
            for cp in row_copies(ch - 1, bb):
                cp.wait()

            @pl.when(ch >= 3)
            def _drain_prev():
                m_scatter(ch - 3, bb).wait()

            compute(ch - 1, bb)

        @pl.when(ch + 1 < NCHUNK)
        def _prefetch_idx():
            idx_copy(ch + 1).start()

        return carry

    lax.fori_loop(0, NCHUNK + 1, step, 0)
    for cc in (NCHUNK - 2, NCHUNK - 1):
        m_scatter(cc, cc % 2).wait()
    plsc.subcore_barrier()
    pltpu.sync_copy(h_sh.at[pl.ds(r0, RSTEP)],
                    hp_hbm.at[c, pl.ds(r0, RSTEP)])


_p2 = pl.kernel(
    _p2_body,
    out_type=jax.ShapeDtypeStruct((NC, N, D), jnp.float32),
    mesh=_sc_mesh,
    compiler_params=pltpu.CompilerParams(needs_layout_passes=False, use_tc_tiling_on_sc=False),
    scratch_types=[
        pltpu.VMEM((4, 2, C), jnp.int32),
        pltpu.VMEM((2, C, D), jnp.float32),
        pltpu.VMEM((2, C, 16), jnp.float32),
        pltpu.VMEM((2, C, D), jnp.float32),
        pltpu.SemaphoreType.DMA((2,)),
        pltpu.SemaphoreType.DMA((2,)),
        pltpu.SemaphoreType.DMA((2,)),
        pltpu.VMEM_SHARED((N, D), jnp.float32),
    ],
)


def kernel(x, edge_index, edge_attr, ck_w, ck_b, qw_w, qw_b, vw_w, vw_b,
           aw_w, aw_b, cw_w, cw_b, lx_w, lx_b, la1_w, la1_b, la2_w, la2_b):
    z16 = jnp.zeros((N, 16), jnp.float32)
    z128 = jnp.zeros((N, D), jnp.float32)

    eas = _ea_prep(edge_attr,
                   ck_w[0, :D], ck_b[0].reshape(1, D),
                   ck_w[1, :D], ck_b[1].reshape(1, D),
                   ck_w[2, :D], ck_b[2].reshape(1, D))

    for l in range(L):
        q, xk, v, lxv = _prep(x,
                              qw_w[l], qw_b[l].reshape(1, D),
                              ck_w[l, D:],
                              vw_w[l], vw_b[l].reshape(1, D),
                              lx_w[l], lx_b[l].reshape(1, D))
        awsp = jnp.broadcast_to(aw_w[l][:, :, None], (H, H, LANES))
        absp = jnp.broadcast_to(aw_b[l][:, None], (H, LANES))
        a_e, scp = _p1(q, xk, eas[l], edge_index, awsp, absp, z16)
        hp = _p2(v, a_e, edge_index, z128)
        x = _node(x, lxv, hp, scp,
                  cw_w[l], cw_b[l].reshape(1, D),
                  la1_w[l, :D], la1_w[l, D:], la1_b[l].reshape(1, D),
                  la2_w[l, :, 0].reshape(1, D), la2_w[l, :, 1].reshape(1, D),
                  la2_b[l, 0].reshape(1, 1), la2_b[l, 1].reshape(1, 1))
    return x


# speedup vs baseline: 3.7023x; 1.0156x over previous
import functools
import math

import jax
import jax.numpy as jnp
from jax import lax
from jax.experimental import pallas as pl
from jax.experimental.pallas import tpu as pltpu
from jax.experimental.pallas import tpu_sc as plsc

N = 10000
E = 320000
D = 128
H = 8
L = 3
LANES = 16
NC = 2
NS = 16
NW = NC * NS
EPW = E // NW
C = 80
NCHUNK = EPW // C
GRP = C // LANES
RSTEP = 632

INV_SQRT_D = 1.0 / math.sqrt(D)
BN = 2000
BE = 4000


def _prep_body(x_ref, wq, bq, wk, wv, bv, wl, bl, q_ref, xk_ref, v_ref, lx_ref):
    xb = x_ref[...]
    q_ref[...] = jnp.dot(xb, wq[...], preferred_element_type=jnp.float32) + bq[...]
    xk_ref[...] = jnp.dot(xb, wk[...], preferred_element_type=jnp.float32) * INV_SQRT_D
    v_ref[...] = jnp.dot(xb, wv[...], preferred_element_type=jnp.float32) + bv[...]
    lx_ref[...] = jnp.dot(xb, wl[...], preferred_element_type=jnp.float32) + bl[...]


_prep = pl.pallas_call(
    _prep_body,
    grid=(N // BN,),
    in_specs=[
        pl.BlockSpec((BN, D), lambda i: (i, 0)),
        pl.BlockSpec((D, D), lambda i: (0, 0)),
        pl.BlockSpec((1, D), lambda i: (0, 0)),
        pl.BlockSpec((D, D), lambda i: (0, 0)),
        pl.BlockSpec((D, D), lambda i: (0, 0)),
        pl.BlockSpec((1, D), lambda i: (0, 0)),
        pl.BlockSpec((D, D), lambda i: (0, 0)),
        pl.BlockSpec((1, D), lambda i: (0, 0)),
    ],
    out_specs=[pl.BlockSpec((BN, D), lambda i: (i, 0))] * 4,
    out_shape=[jax.ShapeDtypeStruct((N, D), jnp.float32)] * 4,
)


def _ea_body(ea_ref, w0, b0, w1, b1, w2, b2, o0, o1, o2):
    eb = ea_ref[...]
    o0[...] = (jnp.dot(eb, w0[...], preferred_element_type=jnp.float32) + b0[...]) * INV_SQRT_D
    o1[...] = (jnp.dot(eb, w1[...], preferred_element_type=jnp.float32) + b1[...]) * INV_SQRT_D
    o2[...] = (jnp.dot(eb, w2[...], preferred_element_type=jnp.float32) + b2[...]) * INV_SQRT_D


_ea_prep = pl.pallas_call(
    _ea_body,
    grid=(E // BE,),
    in_specs=[pl.BlockSpec((BE, D), lambda i: (i, 0))]
    + [pl.BlockSpec((D, D), lambda i: (0, 0)), pl.BlockSpec((1, D), lambda i: (0, 0))] * 3,
    out_specs=[pl.BlockSpec((BE, D), lambda i: (i, 0))] * 3,
    out_shape=[jax.ShapeDtypeStruct((E, D), jnp.float32)] * 3,
)


def _node_body(x_ref, lx_ref, hp_ref, scp_ref, cww, cwb, a1a, a1b, a1bias,
               w20, w21, b20, b21, out_ref):
    xb = x_ref[...]
    hraw = hp_ref[0] + hp_ref[1]
    sc = scp_ref[0] + scp_ref[1]
    deg = sc[:, 8:9]
    inv_sc = 1.0 / jnp.where(sc == 0.0, 1.0, sc)
    hpre = jnp.concatenate(
        [hraw[:, hh * LANES:(hh + 1) * LANES] * inv_sc[:, hh:hh + 1]
         for hh in range(H)], axis=1)
    h = jnp.dot(hpre, cww[...], preferred_element_type=jnp.float32) + deg * cwb[...]
    z = (jnp.dot(lx_ref[...], a1a[...], preferred_element_type=jnp.float32)
         + jnp.dot(h, a1b[...], preferred_element_type=jnp.float32) + a1bias[...])
    z = jnp.where(z > 0, z, 0.2 * z)
    p0 = jnp.sum(z * w20[...], axis=1, keepdims=True) + b20[...]
    p1 = jnp.sum(z * w21[...], axis=1, keepdims=True) + b21[...]
    m = jnp.maximum(p0, p1)
    e0 = jnp.exp(p0 - m)
    e1 = jnp.exp(p1 - m)
    inv = 1.0 / (e0 + e1)
    out_ref[...] = xb * (e0 * inv) + h * (e1 * inv)


_node = pl.pallas_call(
    _node_body,
    grid=(N // BN,),
    in_specs=[
        pl.BlockSpec((BN, D), lambda i: (i, 0)),
        pl.BlockSpec((BN, D), lambda i: (i, 0)),
        pl.BlockSpec((NC, BN, D), lambda i: (0, i, 0)),
        pl.BlockSpec((NC, BN, 16), lambda i: (0, i, 0)),
        pl.BlockSpec((D, D), lambda i: (0, 0)),
        pl.BlockSpec((1, D), lambda i: (0, 0)),
        pl.BlockSpec((D, D), lambda i: (0, 0)),
        pl.BlockSpec((D, D), lambda i: (0, 0)),
        pl.BlockSpec((1, D), lambda i: (0, 0)),
        pl.BlockSpec((1, D), lambda i: (0, 0)),
        pl.BlockSpec((1, D), lambda i: (0, 0)),
        pl.BlockSpec((1, 1), lambda i: (0, 0)),
        pl.BlockSpec((1, 1), lambda i: (0, 0)),
    ],
    out_specs=pl.BlockSpec((BN, D), lambda i: (i, 0)),
    out_shape=jax.ShapeDtypeStruct((N, D), jnp.float32),
)


_sc_mesh = plsc.VectorSubcoreMesh(core_axis_name="c", subcore_axis_name="s")


def _p1_body(q_hbm, xk_hbm, ea_hbm, ei_hbm, awsp_hbm, absp_hbm, z16_hbm,
             a_hbm, scp_hbm,
             sdidx, qrows, krows, erows, abuf, awsp_v, absp_v,
             sem_i, sem_r, sem_a, sem_s, sc_sh):
    c = lax.axis_index("c")
    s = lax.axis_index("s")
    wid = c * NS + s
    e0 = wid * EPW
    r0 = jnp.minimum(s * RSTEP, N - RSTEP)
    pltpu.sync_copy(z16_hbm.at[pl.ds(r0, RSTEP)], sc_sh.at[pl.ds(r0, RSTEP)])
    pltpu.sync_copy(awsp_hbm, awsp_v)
    pltpu.sync_copy(absp_hbm, absp_v)
    plsc.subcore_barrier()

    iota = lax.iota(jnp.int32, LANES)

    def idx_copy(ch):
        bi = lax.rem(ch, 4)
        base = e0 + ch * C
        return pltpu.make_async_copy(ei_hbm.at[:, pl.ds(base, C)], sdidx.at[bi],
                                     sem_i.at[lax.rem(ch, 2)])

    def row_copies(ch, b):
        bi = lax.rem(ch, 4)
        base = e0 + ch * C
        return (
            pltpu.make_async_copy(q_hbm.at[sdidx.at[bi, 1]], qrows.at[b], sem_r.at[b]),
            pltpu.make_async_copy(xk_hbm.at[sdidx.at[bi, 0]], krows.at[b], sem_r.at[b]),
            pltpu.make_async_copy(ea_hbm.at[pl.ds(base, C)], erows.at[b], sem_r.at[b]),
        )

    def a_copy(ch, b):
        base = e0 + ch * C
        return pltpu.make_async_copy(abuf.at[b], a_hbm.at[pl.ds(base, C)],
                                     sem_a.at[b])

    def sc_scatter(ch, b):
        bi = lax.rem(ch, 4)
        return pltpu.make_async_copy(abuf.at[b], sc_sh.at[sdidx.at[bi, 1]],
                                     sem_s.at[b])

    def compute(ch, b):
        qr = qrows.at[b]
        kr = krows.at[b]
        er = erows.at[b]

        def group(g, carry2):
            rows = g * LANES + iota
            ls = []
            for h in range(H):
                acc = [jnp.zeros((LANES,), jnp.float32) for _ in range(4)]
                for dd in range(LANES):
                    col = h * LANES + ((dd + iota) & (LANES - 1))
                    qv = plsc.load_gather(qr, [rows, col])
                    kv = plsc.load_gather(kr, [rows, col])
                    ev = plsc.load_gather(er, [rows, col])
                    acc[dd % 4] = acc[dd % 4] + qv * (kv + ev)
                accs = (acc[0] + acc[1]) + (acc[2] + acc[3])
                ls.append(jnp.where(accs > 0, accs, 0.2 * accs))
            for j in range(H):
                t = absp_v[j]
                for h in range(H):
                    t = t + ls[h] * awsp_v[h, j]
                aj = jnp.exp(t)
                plsc.store_scatter(abuf.at[b], [rows, jnp.full((LANES,), j, jnp.int32)], aj)
            plsc.store_scatter(abuf.at[b], [rows, jnp.full((LANES,), H, jnp.int32)],
                               jnp.ones((LANES,), jnp.float32))
            return carry2

        lax.fori_loop(0, GRP, group, 0)
        a_copy(ch, b).start()
        sc_scatter(ch, b).start(add=True)

    idx_copy(0).start()

    def step(ch, carry):
        b = lax.rem(ch, 2)
        bb = 1 - b

        @pl.when(ch < NCHUNK)
        def _fetch():
            idx_copy(ch).wait()
            for cp in row_copies(ch, b):
                cp.start()

        @pl.when(ch >= 1)
        def _work():
            for cp in row_copies(ch - 1, bb):
                cp.wait()

            @pl.when(ch >= 3)
            def _drain_prev():
                a_copy(ch - 3, bb).wait()
                sc_scatter(ch - 3, bb).wait()

            compute(ch - 1, bb)

        @pl.when(ch + 1 < NCHUNK)
        def _prefetch_idx():
            idx_copy(ch + 1).start()

        return carry

    lax.fori_loop(0, NCHUNK + 1, step, 0)
    for cc in (NCHUNK - 2, NCHUNK - 1):
        a_copy(cc, cc % 2).wait()
        sc_scatter(cc, cc % 2).wait()
    plsc.subcore_barrier()
    pltpu.sync_copy(sc_sh.at[pl.ds(r0, RSTEP)],
                    scp_hbm.at[c, pl.ds(r0, RSTEP)])


_p1 = pl.kernel(
    _p1_body,
    out_type=[jax.ShapeDtypeStruct((E, 16), jnp.float32),
              jax.ShapeDtypeStruct((NC, N, 16), jnp.float32)],
    mesh=_sc_mesh,
    compiler_params=pltpu.CompilerParams(needs_layout_passes=False, use_tc_tiling_on_sc=False),
    scratch_types=[
        pltpu.VMEM((4, 2, C), jnp.int32),
        pltpu.VMEM((2, C, D), jnp.float32),
        pltpu.VMEM((2, C, D), jnp.float32),
        pltpu.VMEM((2, C, D), jnp.float32),
        pltpu.VMEM((2, C, 16), jnp.float32),
        pltpu.VMEM((H, H, LANES), jnp.float32),
        pltpu.VMEM((H, LANES), jnp.float32),
        pltpu.SemaphoreType.DMA((2,)),
        pltpu.SemaphoreType.DMA((2,)),
        pltpu.SemaphoreType.DMA((2,)),
        pltpu.SemaphoreType.DMA((2,)),
        pltpu.VMEM_SHARED((N, 16), jnp.float32),
    ],
)


def _p2_body(v_hbm, a_hbm, ei_hbm, z128_hbm,
             hp_hbm,
             sdidx, vrows, arows, mbuf, sem_i, sem_r, sem_s, h_sh):
    c = lax.axis_index("c")
    s = lax.axis_index("s")
    wid = c * NS + s
    e0 = wid * EPW
    r0 = jnp.minimum(s * RSTEP, N - RSTEP)
    pltpu.sync_copy(z128_hbm.at[pl.ds(r0, RSTEP)], h_sh.at[pl.ds(r0, RSTEP)])
    plsc.subcore_barrier()

    iota = lax.iota(jnp.int32, LANES)

    def idx_copy(ch):
        bi = lax.rem(ch, 4)
        base = e0 + ch * C
        return pltpu.make_async_copy(ei_hbm.at[:, pl.ds(base, C)], sdidx.at[bi],
                                     sem_i.at[lax.rem(ch, 2)])

    def row_copies(ch, b):
        bi = lax.rem(ch, 4)
        base = e0 + ch * C
        return (
            pltpu.make_async_copy(v_hbm.at[sdidx.at[bi, 0]], vrows.at[b], sem_r.at[b]),
            pltpu.make_async_copy(a_hbm.at[pl.ds(base, C)], arows.at[b], sem_r.at[b]),
        )

    def m_scatter(ch, b):
        bi = lax.rem(ch, 4)
        return pltpu.make_async_copy(mbuf.at[b], h_sh.at[sdidx.at[bi, 1]],
                                     sem_s.at[b])

    def compute(ch, b):
        vr = vrows.at[b]
        ar = arows.at[b]

        def group(g, carry2):
            rows = g * LANES + iota
            for h in range(H):
                hcol = jnp.full((LANES,), h, jnp.int32)
                attn = plsc.load_gather(ar, [rows, hcol])
                for dd in range(LANES):
                    col = h * LANES + ((dd + iota) & (LANES - 1))
                    mv = plsc.load_gather(vr, [rows, col])
                    plsc.store_scatter(mbuf.at[b], [rows, col], attn * mv)
            return carry2

        lax.fori_loop(0, GRP, group, 0)
        m_scatter(ch, b).start(add=True)

    idx_copy(0).start()

    def step(ch, carry):
        b = lax.rem(ch, 2)
        bb = 1 - b

        @pl.when(ch < NCHUNK)
        def _fetch():
            idx_copy(ch).wait()
            for cp in row_copies(ch, b):
                cp.start()

        @pl.when(ch >= 1)
        def _work():
            for cp in row_copies(ch - 1, bb):
                cp.wait()

            @pl.when(ch >= 3)
            def _drain_prev():
                m_scatter(ch - 3, bb).wait()

            compute(ch - 1, bb)

        @pl.when(ch + 1 < NCHUNK)
        def _prefetch_idx():
            idx_copy(ch + 1).start()

        return carry

    lax.fori_loop(0, NCHUNK + 1, step, 0)
    for cc in (NCHUNK - 2, NCHUNK - 1):
        m_scatter(cc, cc % 2).wait()
    plsc.subcore_barrier()
    pltpu.sync_copy(h_sh.at[pl.ds(r0, RSTEP)],
                    hp_hbm.at[c, pl.ds(r0, RSTEP)])


_p2 = pl.kernel(
    _p2_body,
    out_type=jax.ShapeDtypeStruct((NC, N, D), jnp.float32),
    mesh=_sc_mesh,
    compiler_params=pltpu.CompilerParams(needs_layout_passes=False, use_tc_tiling_on_sc=False),
    scratch_types=[
        pltpu.VMEM((4, 2, C), jnp.int32),
        pltpu.VMEM((2, C, D), jnp.float32),
        pltpu.VMEM((2, C, 16), jnp.float32),
        pltpu.VMEM((2, C, D), jnp.float32),
        pltpu.SemaphoreType.DMA((2,)),
        pltpu.SemaphoreType.DMA((2,)),
        pltpu.SemaphoreType.DMA((2,)),
        pltpu.VMEM_SHARED((N, D), jnp.float32),
    ],
)


def kernel(x, edge_index, edge_attr, ck_w, ck_b, qw_w, qw_b, vw_w, vw_b,
           aw_w, aw_b, cw_w, cw_b, lx_w, lx_b, la1_w, la1_b, la2_w, la2_b):
    z16 = jnp.zeros((N, 16), jnp.float32)
    z128 = jnp.zeros((N, D), jnp.float32)

    eas = _ea_prep(edge_attr,
                   ck_w[0, :D], ck_b[0].reshape(1, D),
                   ck_w[1, :D], ck_b[1].reshape(1, D),
                   ck_w[2, :D], ck_b[2].reshape(1, D))

    for l in range(L):
        q, xk, v, lxv = _prep(x,
                              qw_w[l], qw_b[l].reshape(1, D),
                              ck_w[l, D:],
                              vw_w[l], vw_b[l].reshape(1, D),
                              lx_w[l], lx_b[l].reshape(1, D))
        awsp = jnp.broadcast_to(aw_w[l][:, :, None], (H, H, LANES))
        absp = jnp.broadcast_to(aw_b[l][:, None], (H, LANES))
        a_e, scp = _p1(q, xk, eas[l], edge_index, awsp, absp, z16)
        hp = _p2(v, a_e, edge_index, z128)
        x = _node(x, lxv, hp, scp,
                  cw_w[l], cw_b[l].reshape(1, D),
                  la1_w[l, :D], la1_w[l, D:], la1_b[l].reshape(1, D),
                  la2_w[l, :, 0].reshape(1, D), la2_w[l, :, 1].reshape(1, D),
                  la2_b[l, 0].reshape(1, 1), la2_b[l, 1].reshape(1, 1))
    return x


# speedup vs baseline: 3.8575x; 1.0419x over previous
import functools
import math

import jax
import jax.numpy as jnp
from jax import lax
from jax.experimental import pallas as pl
from jax.experimental.pallas import tpu as pltpu
from jax.experimental.pallas import tpu_sc as plsc

N = 10000
E = 320000
D = 128
H = 8
L = 3
LANES = 16
NC = 2
NS = 16
NW = NC * NS
EPW = E // NW
C = 80
NCHUNK = EPW // C
GRP = C // LANES
RSTEP = 632

INV_SQRT_D = 1.0 / math.sqrt(D)
BN = 2000
BE = 4000


def _prep_body(x_ref, wq, bq, wk, wv, bv, wl, bl, q_ref, xk_ref, v_ref, lx_ref):
    xb = x_ref[...]
    q_ref[...] = jnp.dot(xb, wq[...], preferred_element_type=jnp.float32) + bq[...]
    xk_ref[...] = jnp.dot(xb, wk[...], preferred_element_type=jnp.float32) * INV_SQRT_D
    v_ref[...] = jnp.dot(xb, wv[...], preferred_element_type=jnp.float32) + bv[...]
    lx_ref[...] = jnp.dot(xb, wl[...], preferred_element_type=jnp.float32) + bl[...]


_prep = pl.pallas_call(
    _prep_body,
    grid=(N // BN,),
    in_specs=[
        pl.BlockSpec((BN, D), lambda i: (i, 0)),
        pl.BlockSpec((D, D), lambda i: (0, 0)),
        pl.BlockSpec((1, D), lambda i: (0, 0)),
        pl.BlockSpec((D, D), lambda i: (0, 0)),
        pl.BlockSpec((D, D), lambda i: (0, 0)),
        pl.BlockSpec((1, D), lambda i: (0, 0)),
        pl.BlockSpec((D, D), lambda i: (0, 0)),
        pl.BlockSpec((1, D), lambda i: (0, 0)),
    ],
    out_specs=[pl.BlockSpec((BN, D), lambda i: (i, 0))] * 4,
    out_shape=[jax.ShapeDtypeStruct((N, D), jnp.float32)] * 4,
)


def _ea_body(ea_ref, w0, b0, w1, b1, w2, b2, o0, o1, o2):
    eb = ea_ref[...]
    o0[...] = (jnp.dot(eb, w0[...], preferred_element_type=jnp.float32) + b0[...]) * INV_SQRT_D
    o1[...] = (jnp.dot(eb, w1[...], preferred_element_type=jnp.float32) + b1[...]) * INV_SQRT_D
    o2[...] = (jnp.dot(eb, w2[...], preferred_element_type=jnp.float32) + b2[...]) * INV_SQRT_D


_ea_prep = pl.pallas_call(
    _ea_body,
    grid=(E // BE,),
    in_specs=[pl.BlockSpec((BE, D), lambda i: (i, 0))]
    + [pl.BlockSpec((D, D), lambda i: (0, 0)), pl.BlockSpec((1, D), lambda i: (0, 0))] * 3,
    out_specs=[pl.BlockSpec((BE, D), lambda i: (i, 0))] * 3,
    out_shape=[jax.ShapeDtypeStruct((E, D), jnp.float32)] * 3,
)


def _node_body(x_ref, lx_ref, hp_ref, scp_ref, cww, cwb, a1a, a1b, a1bias,
               w20, w21, b20, b21, out_ref):
    xb = x_ref[...]
    hraw = hp_ref[0] + hp_ref[1]
    sc = scp_ref[0] + scp_ref[1]
    deg = sc[:, 8:9]
    inv_sc = 1.0 / jnp.where(sc == 0.0, 1.0, sc)
    hpre = jnp.concatenate(
        [hraw[:, hh * LANES:(hh + 1) * LANES] * inv_sc[:, hh:hh + 1]
         for hh in range(H)], axis=1)
    h = jnp.dot(hpre, cww[...], preferred_element_type=jnp.float32) + deg * cwb[...]
    z = (jnp.dot(lx_ref[...], a1a[...], preferred_element_type=jnp.float32)
         + jnp.dot(h, a1b[...], preferred_element_type=jnp.float32) + a1bias[...])
    z = jnp.where(z > 0, z, 0.2 * z)
    p0 = jnp.sum(z * w20[...], axis=1, keepdims=True) + b20[...]
    p1 = jnp.sum(z * w21[...], axis=1, keepdims=True) + b21[...]
    m = jnp.maximum(p0, p1)
    e0 = jnp.exp(p0 - m)
    e1 = jnp.exp(p1 - m)
    inv = 1.0 / (e0 + e1)
    out_ref[...] = xb * (e0 * inv) + h * (e1 * inv)


_node = pl.pallas_call(
    _node_body,
    grid=(N // BN,),
    in_specs=[
        pl.BlockSpec((BN, D), lambda i: (i, 0)),
        pl.BlockSpec((BN, D), lambda i: (i, 0)),
        pl.BlockSpec((NC, BN, D), lambda i: (0, i, 0)),
        pl.BlockSpec((NC, BN, 16), lambda i: (0, i, 0)),
        pl.BlockSpec((D, D), lambda i: (0, 0)),
        pl.BlockSpec((1, D), lambda i: (0, 0)),
        pl.BlockSpec((D, D), lambda i: (0, 0)),
        pl.BlockSpec((D, D), lambda i: (0, 0)),
        pl.BlockSpec((1, D), lambda i: (0, 0)),
        pl.BlockSpec((1, D), lambda i: (0, 0)),
        pl.BlockSpec((1, D), lambda i: (0, 0)),
        pl.BlockSpec((1, 1), lambda i: (0, 0)),
        pl.BlockSpec((1, 1), lambda i: (0, 0)),
    ],
    out_specs=pl.BlockSpec((BN, D), lambda i: (i, 0)),
    out_shape=jax.ShapeDtypeStruct((N, D), jnp.float32),
)


_sc_mesh = plsc.VectorSubcoreMesh(core_axis_name="c", subcore_axis_name="s")


def _p1_body(q_hbm, xk_hbm, ea_hbm, ei4_hbm, awsp_hbm, absp_hbm, z16_hbm,
             a_hbm, scp_hbm,
             sdidx, qrows, krows, erows, abuf, awsp_v, absp_v,
             sem_r, sem_a, sem_s, sc_sh):
    c = lax.axis_index("c")
    s = lax.axis_index("s")
    wid = c * NS + s
    e0 = wid * EPW
    r0 = jnp.minimum(s * RSTEP, N - RSTEP)
    pltpu.sync_copy(z16_hbm.at[pl.ds(r0, RSTEP)], sc_sh.at[pl.ds(r0, RSTEP)])
    pltpu.sync_copy(awsp_hbm, awsp_v)
    pltpu.sync_copy(absp_hbm, absp_v)
    pltpu.sync_copy(ei4_hbm.at[:, wid], sdidx)
    plsc.subcore_barrier()

    iota = lax.iota(jnp.int32, LANES)

    def row_copies(ch, b):
        base = e0 + ch * C
        return (
            pltpu.make_async_copy(q_hbm.at[sdidx.at[1, ch]], qrows.at[b], sem_r.at[b]),
            pltpu.make_async_copy(xk_hbm.at[sdidx.at[0, ch]], krows.at[b], sem_r.at[b]),
            pltpu.make_async_copy(ea_hbm.at[pl.ds(base, C)], erows.at[b], sem_r.at[b]),
        )

    def a_copy(ch, b):
        base = e0 + ch * C
        return pltpu.make_async_copy(abuf.at[b], a_hbm.at[pl.ds(base, C)],
                                     sem_a.at[b])

    def sc_scatter(ch, b):
        return pltpu.make_async_copy(abuf.at[b], sc_sh.at[sdidx.at[1, ch]],
                                     sem_s.at[b])

    def compute(ch, b):
        qr = qrows.at[b]
        kr = krows.at[b]
        er = erows.at[b]

        def group(g, carry2):
            rows = g * LANES + iota
            ls = []
            for h in range(H):
                acc = [jnp.zeros((LANES,), jnp.float32) for _ in range(4)]
                for dd in range(LANES):
                    col = h * LANES + ((dd + iota) & (LANES - 1))
                    qv = plsc.load_gather(qr, [rows, col])
                    kv = plsc.load_gather(kr, [rows, col])
                    ev = plsc.load_gather(er, [rows, col])
                    acc[dd % 4] = acc[dd % 4] + qv * (kv + ev)
                accs = (acc[0] + acc[1]) + (acc[2] + acc[3])
                ls.append(jnp.where(accs > 0, accs, 0.2 * accs))
            for j in range(H):
                t = absp_v[j]
                for h in range(H):
                    t = t + ls[h] * awsp_v[h, j]
                aj = jnp.exp(t)
                plsc.store_scatter(abuf.at[b], [rows, jnp.full((LANES,), j, jnp.int32)], aj)
            plsc.store_scatter(abuf.at[b], [rows, jnp.full((LANES,), H, jnp.int32)],
                               jnp.ones((LANES,), jnp.float32))
            return carry2

        lax.fori_loop(0, GRP, group, 0)
        a_copy(ch, b).start()
        sc_scatter(ch, b).start(add=True)

    def step(ch, carry):
        b = lax.rem(ch, 2)
        bb = 1 - b

        @pl.when(ch < NCHUNK)
        def _fetch():
            for cp in row_copies(ch, b):
                cp.start()

        @pl.when(ch >= 1)
        def _work():
            for cp in row_copies(ch - 1, bb):
                cp.wait()

            @pl.when(ch >= 3)
            def _drain_prev():
                a_copy(ch - 3, bb).wait()
                sc_scatter(ch - 3, bb).wait()

            compute(ch - 1, bb)

        return carry

    lax.fori_loop(0, NCHUNK + 1, step, 0)
    for cc in (NCHUNK - 2, NCHUNK - 1):
        a_copy(cc, cc % 2).wait()
        sc_scatter(cc, cc % 2).wait()
    plsc.subcore_barrier()
    pltpu.sync_copy(sc_sh.at[pl.ds(r0, RSTEP)],
                    scp_hbm.at[c, pl.ds(r0, RSTEP)])


_p1 = pl.kernel(
    _p1_body,
    out_type=[jax.ShapeDtypeStruct((E, 16), jnp.float32),
              jax.ShapeDtypeStruct((NC, N, 16), jnp.float32)],
    mesh=_sc_mesh,
    compiler_params=pltpu.CompilerParams(needs_layout_passes=False, use_tc_tiling_on_sc=False),
    scratch_types=[
        pltpu.VMEM((2, NCHUNK, C), jnp.int32),
        pltpu.VMEM((2, C, D), jnp.float32),
        pltpu.VMEM((2, C, D), jnp.float32),
        pltpu.VMEM((2, C, D), jnp.float32),
        pltpu.VMEM((2, C, 16), jnp.float32),
        pltpu.VMEM((H, H, LANES), jnp.float32),
        pltpu.VMEM((H, LANES), jnp.float32),
        pltpu.SemaphoreType.DMA((2,)),
        pltpu.SemaphoreType.DMA((2,)),
        pltpu.SemaphoreType.DMA((2,)),
        pltpu.VMEM_SHARED((N, 16), jnp.float32),
    ],
)


def _p2_body(v_hbm, a_hbm, ei_hbm, z128_hbm,
             hp_hbm,
             sdidx, vrows, arows, mbuf, sem_i, sem_r, sem_s, h_sh):
    c = lax.axis_index("c")
    s = lax.axis_index("s")
    wid = c * NS + s
    e0 = wid * EPW
    r0 = jnp.minimum(s * RSTEP, N - RSTEP)
    pltpu.sync_copy(z128_hbm.at[pl.ds(r0, RSTEP)], h_sh.at[pl.ds(r0, RSTEP)])
    plsc.subcore_barrier()

    iota = lax.iota(jnp.int32, LANES)

    def idx_copy(ch):
        bi = lax.rem(ch, 4)
        base = e0 + ch * C
        return pltpu.make_async_copy(ei_hbm.at[:, pl.ds(base, C)], sdidx.at[bi],
                                     sem_i.at[lax.rem(ch, 2)])

    def row_copies(ch, b):
        bi = lax.rem(ch, 4)
        base = e0 + ch * C
        return (
            pltpu.make_async_copy(v_hbm.at[sdidx.at[bi, 0]], vrows.at[b], sem_r.at[b]),
            pltpu.make_async_copy(a_hbm.at[pl.ds(base, C)], arows.at[b], sem_r.at[b]),
        )

    def m_scatter(ch, b):
        bi = lax.rem(ch, 4)
        return pltpu.make_async_copy(mbuf.at[b], h_sh.at[sdidx.at[bi, 1]],
                                     sem_s.at[b])

    def compute(ch, b):
        vr = vrows.at[b]
        ar = arows.at[b]

        def group(g, carry2):
            rows = g * LANES + iota
            for h in range(H):
                hcol = jnp.full((LANES,), h, jnp.int32)
                attn = plsc.load_gather(ar, [rows, hcol])
                for dd in range(LANES):
                    col = h * LANES + ((dd + iota) & (LANES - 1))
                    mv = plsc.load_gather(vr, [rows, col])
                    plsc.store_scatter(mbuf.at[b], [rows, col], attn * mv)
            return carry2

        lax.fori_loop(0, GRP, group, 0)
        m_scatter(ch, b).start(add=True)

    idx_copy(0).start()

    def step(ch, carry):
        b = lax.rem(ch, 2)
        bb = 1 - b

        @pl.when(ch < NCHUNK)
        def _fetch():
            idx_copy(ch).wait()
            for cp in row_copies(ch, b):
                cp.start()

        @pl.when(ch >= 1)
        def _work():
            for cp in row_copies(ch - 1, bb):
                cp.wait()

            @pl.when(ch >= 3)
            def _drain_prev():
                m_scatter(ch - 3, bb).wait()

            compute(ch - 1, bb)

        @pl.when(ch + 1 < NCHUNK)
        def _prefetch_idx():
            idx_copy(ch + 1).start()

        return carry

    lax.fori_loop(0, NCHUNK + 1, step, 0)
    for cc in (NCHUNK - 2, NCHUNK - 1):
        m_scatter(cc, cc % 2).wait()
    plsc.subcore_barrier()
    pltpu.sync_copy(h_sh.at[pl.ds(r0, RSTEP)],
                    hp_hbm.at[c, pl.ds(r0, RSTEP)])


_p2 = pl.kernel(
    _p2_body,
    out_type=jax.ShapeDtypeStruct((NC, N, D), jnp.float32),
    mesh=_sc_mesh,
    compiler_params=pltpu.CompilerParams(needs_layout_passes=False, use_tc_tiling_on_sc=False),
    scratch_types=[
        pltpu.VMEM((4, 2, C), jnp.int32),
        pltpu.VMEM((2, C, D), jnp.float32),
        pltpu.VMEM((2, C, 16), jnp.float32),
        pltpu.VMEM((2, C, D), jnp.float32),
        pltpu.SemaphoreType.DMA((2,)),
        pltpu.SemaphoreType.DMA((2,)),
        pltpu.SemaphoreType.DMA((2,)),
        pltpu.VMEM_SHARED((N, D), jnp.float32),
    ],
)


def kernel(x, edge_index, edge_attr, ck_w, ck_b, qw_w, qw_b, vw_w, vw_b,
           aw_w, aw_b, cw_w, cw_b, lx_w, lx_b, la1_w, la1_b, la2_w, la2_b):
    ei4 = edge_index.reshape(2, NW, NCHUNK, C)
    z16 = jnp.zeros((N, 16), jnp.float32)
    z128 = jnp.zeros((N, D), jnp.float32)

    eas = _ea_prep(edge_attr,
                   ck_w[0, :D], ck_b[0].reshape(1, D),
                   ck_w[1, :D], ck_b[1].reshape(1, D),
                   ck_w[2, :D], ck_b[2].reshape(1, D))

    for l in range(L):
        q, xk, v, lxv = _prep(x,
                              qw_w[l], qw_b[l].reshape(1, D),
                              ck_w[l, D:],
                              vw_w[l], vw_b[l].reshape(1, D),
                              lx_w[l], lx_b[l].reshape(1, D))
        awsp = jnp.broadcast_to(aw_w[l][:, :, None], (H, H, LANES))
        absp = jnp.broadcast_to(aw_b[l][:, None], (H, LANES))
        a_e, scp = _p1(q, xk, eas[l], ei4, awsp, absp, z16)
        hp = _p2(v, a_e, edge_index, z128)
        x = _node(x, lxv, hp, scp,
                  cw_w[l], cw_b[l].reshape(1, D),
                  la1_w[l, :D], la1_w[l, D:], la1_b[l].reshape(1, D),
                  la2_w[l, :, 0].reshape(1, D), la2_w[l, :, 1].reshape(1, D),
                  la2_b[l, 0].reshape(1, 1), la2_b[l, 1].reshape(1, 1))
    return x
